# baseline (speedup 1.0000x reference)
import functools

import jax
import jax.numpy as jnp
import numpy as np
from jax import lax
from jax.experimental import pallas as pl
from jax.experimental.pallas import tpu as pltpu
from jax.experimental.pallas import tpu_sc as plsc

N = 10000
E = 320000
H = 128
D = 144
NC = 2
NS = 16
NW = NC * NS
PER_W = E // NW
C = 80
NCH = PER_W // C
ROWS_PER_TILE = N // NS

EB = 1600
NB = 1000

_INV_SQRT_H = float(1.0 / np.sqrt(float(H)))


def _silu(v):
    return v * jax.nn.sigmoid(v)


_sc_mesh = plsc.VectorSubcoreMesh(core_axis_name="c", subcore_axis_name="s")


def _gather_body(rowtab, coltab, ridx, cidx, out_r, out_c,
                 idxr_v, idxc_v, bufr, bufc, semr, semc):
    wid = lax.axis_index("c") * NS + lax.axis_index("s")
    base = wid * PER_W

    def body(ck, carry):
        off = base + ck * C
        pltpu.sync_copy(ridx.at[pl.ds(off, C)], idxr_v)
        pltpu.sync_copy(cidx.at[pl.ds(off, C)], idxc_v)
        cp1 = pltpu.async_copy(rowtab.at[idxr_v], bufr, semr)
        cp2 = pltpu.async_copy(coltab.at[idxc_v], bufc, semc)
        cp1.wait()
        cp2.wait()
        pltpu.sync_copy(bufr, out_r.at[pl.ds(off, C)])
        pltpu.sync_copy(bufc, out_c.at[pl.ds(off, C)])
        return carry

    lax.fori_loop(0, NCH, body, 0)


_sc_gather = pl.kernel(
    _gather_body,
    out_type=[jax.ShapeDtypeStruct((E, D), jnp.float32),
              jax.ShapeDtypeStruct((E, D), jnp.float32)],
    mesh=_sc_mesh,
    compiler_params=pltpu.CompilerParams(use_tc_tiling_on_sc=False),
    scratch_types=[
        pltpu.VMEM((C,), jnp.int32),
        pltpu.VMEM((C,), jnp.int32),
        pltpu.VMEM((C, D), jnp.float32),
        pltpu.VMEM((C, D), jnp.float32),
        pltpu.SemaphoreType.DMA,
        pltpu.SemaphoreType.DMA,
    ],
)


def _scatter_body(payload, cidx, zeros, out, idx_v, pay_v, acc):
    cid = lax.axis_index("c")
    sid = lax.axis_index("s")
    wid = cid * NS + sid
    rows0 = sid * ROWS_PER_TILE
    pltpu.sync_copy(zeros.at[pl.ds(rows0, ROWS_PER_TILE)],
                    acc.at[pl.ds(rows0, ROWS_PER_TILE)])
    plsc.subcore_barrier()
    base = wid * PER_W

    def body(ck, carry):
        off = base + ck * C
        pltpu.sync_copy(cidx.at[pl.ds(off, C)], idx_v)
        pltpu.sync_copy(payload.at[pl.ds(off, C)], pay_v)
        pltpu.sync_copy(pay_v, acc.at[idx_v], add=True)
        return carry

    lax.fori_loop(0, NCH, body, 0)
    plsc.subcore_barrier()
    pltpu.sync_copy(acc.at[pl.ds(rows0, ROWS_PER_TILE)],
                    out.at[pl.ds(cid * N + rows0, ROWS_PER_TILE)])


_sc_scatter = pl.kernel(
    _scatter_body,
    out_type=jax.ShapeDtypeStruct((2 * N, D), jnp.float32),
    mesh=_sc_mesh,
    compiler_params=pltpu.CompilerParams(use_tc_tiling_on_sc=False),
    scratch_types=[
        pltpu.VMEM((C,), jnp.int32),
        pltpu.VMEM((C, D), jnp.float32),
        pltpu.VMEM_SHARED((N, D), jnp.float32),
    ],
)


def _edge_body(ger, gec, eav, wr, br, wc, bc, wle, ble, we2, be2, ws8,
               s1w, s1b, m1w, m1b, wh, bh, out):
    hr = ger[:, :H].astype(jnp.bfloat16)
    hc = gec[:, :H].astype(jnp.bfloat16)
    zr = jnp.dot(hr, wr[...], preferred_element_type=jnp.float32) + br[...]
    zc = jnp.dot(hc, wc[...], preferred_element_type=jnp.float32) + bc[...]
    ea = eav[:, :16]
    e = jnp.dot(ea, wle[...], preferred_element_type=jnp.float32) + ble[...]
    ez = jnp.dot(e, we2[...], preferred_element_type=jnp.float32) + be2[...]
    sig_in = ez[:, :H] + zc[:, :H]
    kk = _silu(jnp.dot(_silu(sig_in).astype(jnp.bfloat16), s1w[...],
                       preferred_element_type=jnp.float32) + s1b[...])
    q = zr[:, :H]
    w = jax.nn.sigmoid(q * kk * _INV_SQRT_H)
    m_in = ez[:, H:] + zr[:, H:2 * H] + zc[:, H:2 * H]
    msg = _silu(jnp.dot(_silu(m_in).astype(jnp.bfloat16), m1w[...],
                        preferred_element_type=jnp.float32) + m1b[...])
    m_ij = w * msg

    ev = eav[:, 16:19]
    pr = ger[:, H:H + 3]
    pc = gec[:, H:H + 3]
    posc = gec[:, H + 3:H + 6]
    s8 = jnp.concatenate([
        jnp.sum(ev * pr, axis=1, keepdims=True),
        jnp.sum(ev * pc, axis=1, keepdims=True),
        jnp.sum(pr * pc, axis=1, keepdims=True),
        jnp.sqrt(jnp.sum(ev * ev, axis=1, keepdims=True)),
        jnp.sqrt(jnp.sum(pr * pr, axis=1, keepdims=True)),
        jnp.sqrt(jnp.sum(pc * pc, axis=1, keepdims=True)),
        jnp.zeros((ev.shape[0], 2), jnp.float32)], axis=1)
    sz = jnp.dot(s8, ws8[...], preferred_element_type=jnp.float32)
    a_in = sz[:, :H] + zr[:, 2 * H:3 * H] + zc[:, 2 * H:3 * H]
    b_in = sz[:, H:2 * H] + zr[:, 3 * H:4 * H] + zc[:, 3 * H:4 * H]
    g_in = sz[:, 2 * H:] + zr[:, 4 * H:] + zc[:, 4 * H:]
    cat3 = jnp.concatenate([_silu(a_in), _silu(b_in), _silu(g_in)], axis=1)
    abg = jnp.dot(cat3, wh[...], preferred_element_type=jnp.float32) + bh[...]
    a = abg[:, 0:3]
    b = jax.nn.sigmoid(abg[:, 3:6])
    g = jnp.tanh(abg[:, 6:9])
    pv = a * (ev * g) + b * posc
    nb = ev.shape[0]
    out[...] = jnp.concatenate(
        [m_ij, pv, jnp.ones((nb, 1), jnp.float32),
         jnp.zeros((nb, D - H - 4), jnp.float32)], axis=1)


def _edge_call(ger, gec, eav, weights):
    full = lambda shape: pl.BlockSpec(shape, lambda i: (0, 0))
    wspecs = [full(w.shape) for w in weights]
    return pl.pallas_call(
        _edge_body,
        grid=(E // EB,),
        in_specs=[pl.BlockSpec((EB, D), lambda i: (i, 0)),
                  pl.BlockSpec((EB, D), lambda i: (i, 0)),
                  pl.BlockSpec((EB, 32), lambda i: (i, 0))] + wspecs,
        out_specs=pl.BlockSpec((EB, D), lambda i: (i, 0)),
        out_shape=jax.ShapeDtypeStruct((E, D), jnp.float32),
    )(ger, gec, eav, *weights)


def _node_body(p0, p1, h, pos, w0, b0, w1, b1, hout, posout):
    acc = p0[...] + p1[...]
    cnt = jnp.maximum(acc[:, H + 3:H + 4], 1.0)
    agg = acc[:, :H] / cnt
    dpos = acc[:, H:H + 3] / cnt
    hcat = jnp.concatenate([agg, h[...]], axis=1)
    u = _silu(jnp.dot(hcat, w0[...], preferred_element_type=jnp.float32)
              + b0[...])
    hout[...] = jnp.dot(u, w1[...], preferred_element_type=jnp.float32) \
        + b1[...] + h[...]
    posout[...] = pos[...] + dpos


def _node_call(parts, h, pos, w0, b0, w1, b1):
    nblk = N // NB
    full = lambda shape: pl.BlockSpec(shape, lambda i: (0, 0))
    return pl.pallas_call(
        _node_body,
        grid=(nblk,),
        in_specs=[pl.BlockSpec((NB, D), lambda i: (i, 0)),
                  pl.BlockSpec((NB, D), lambda i: (i + nblk, 0)),
                  pl.BlockSpec((NB, H), lambda i: (i, 0)),
                  pl.BlockSpec((NB, 3), lambda i: (i, 0)),
                  full(w0.shape), full(b0.shape),
                  full(w1.shape), full(b1.shape)],
        out_specs=[pl.BlockSpec((NB, H), lambda i: (i, 0)),
                   pl.BlockSpec((NB, 3), lambda i: (i, 0))],
        out_shape=[jax.ShapeDtypeStruct((N, H), jnp.float32),
                   jax.ShapeDtypeStruct((N, 3), jnp.float32)],
    )(parts, parts, h, pos, w0, b0, w1, b1)


def _dense_body(x, w, b, out):
    out[...] = jnp.dot(x[...], w[...],
                       preferred_element_type=jnp.float32) + b[...]


def _dense_call(x, w, b):
    n, di = x.shape
    do = w.shape[1]
    full = lambda shape: pl.BlockSpec(shape, lambda i: (0, 0))
    return pl.pallas_call(
        _dense_body,
        grid=(n // NB,),
        in_specs=[pl.BlockSpec((NB, di), lambda i: (i, 0)),
                  full(w.shape), full(b.shape)],
        out_specs=pl.BlockSpec((NB, do), lambda i: (i, 0)),
        out_shape=jax.ShapeDtypeStruct((n, do), jnp.float32),
    )(x, w, b)


def _final_body(h, w, b, hout, pooled):
    i = pl.program_id(0)
    hf = jnp.dot(h[...], w[...], preferred_element_type=jnp.float32) + b[...]
    hout[...] = hf
    s = jnp.sum(hf, axis=0, keepdims=True)

    @pl.when(i == 0)
    def _():
        pooled[...] = s

    @pl.when(i > 0)
    def _():
        pooled[...] += s

    @pl.when(i == (N // NB) - 1)
    def _():
        pooled[...] = pooled[...] * (1.0 / N)


def _final_call(h, w, b):
    full = lambda shape: pl.BlockSpec(shape, lambda i: (0, 0))
    return pl.pallas_call(
        _final_body,
        grid=(N // NB,),
        in_specs=[pl.BlockSpec((NB, H), lambda i: (i, 0)),
                  full(w.shape), full(b.shape)],
        out_specs=[pl.BlockSpec((NB, H), lambda i: (i, 0)),
                   pl.BlockSpec((1, H), lambda i: (0, 0))],
        out_shape=[jax.ShapeDtypeStruct((N, H), jnp.float32),
                   jax.ShapeDtypeStruct((1, H), jnp.float32)],
    )(h, w, b)


def _layer_weights(p):
    Qw, Qb = p["LnQ"]["w"], p["LnQ"]["b"]
    Kw, Kb = p["LnK"]["w"], p["LnK"]["b"]
    Vw, Vb = p["LnV"]["w"], p["LnV"]["b"]
    s0w, s0b = p["sigma0"]["w"], p["sigma0"]["b"]
    s0we, s0wk = s0w[:, :16], s0w[:, 16:]
    m0w, m0b = p["msg0"]["w"], p["msg0"]["b"]
    m0we, m0wr, m0wc = m0w[:, :16], m0w[:, 16:16 + H], m0w[:, 16 + H:]
    a0w, a0b = p["alpha0"]["w"], p["alpha0"]["b"]
    b0w, b0b = p["beta0"]["w"], p["beta0"]["b"]
    g0w, g0b = p["gate0"]["w"], p["gate0"]["b"]
    z = jnp.zeros((3 * H,), jnp.float32)
    W_r = jnp.concatenate([Qw, m0wr @ Vw, a0w[:, 6 + H:], b0w[:, 6 + H:],
                           g0w[:, 6 + H:]], axis=0)
    b_r = jnp.concatenate([Qb, m0wr @ Vb, z], axis=0)
    W_c = jnp.concatenate([s0wk @ Kw, m0wc @ Vw, a0w[:, 6:6 + H],
                           b0w[:, 6:6 + H], g0w[:, 6:6 + H]], axis=0)
    b_c = jnp.concatenate([s0wk @ Kb, m0wc @ Vb, a0b, b0b, g0b], axis=0)
    W_e2 = jnp.concatenate([s0we, m0we], axis=0)
    b_e2 = jnp.concatenate([s0b, m0b], axis=0)
    ws6 = jnp.concatenate([a0w[:, :6], b0w[:, :6], g0w[:, :6]], axis=0)
    W_s8 = jnp.concatenate([ws6, jnp.zeros((3 * H, 2), jnp.float32)], axis=1)
    zh = jnp.zeros((3, H), jnp.float32)
    Wh = jnp.concatenate([
        jnp.concatenate([p["alpha1"]["w"], zh, zh], axis=1),
        jnp.concatenate([zh, p["beta1"]["w"], zh], axis=1),
        jnp.concatenate([zh, zh, p["gate1"]["w"]], axis=1)], axis=0)
    bhd = jnp.concatenate([p["alpha1"]["b"], p["beta1"]["b"],
                           p["gate1"]["b"]], axis=0)
    r1 = lambda v: v.reshape(1, -1)
    bf = lambda v: v.astype(jnp.bfloat16)
    return [bf(W_r.T), r1(b_r), bf(W_c.T), r1(b_c),
            p["LnE"]["w"].T, r1(p["LnE"]["b"]),
            W_e2.T, r1(b_e2), W_s8.T,
            bf(p["sigma1"]["w"].T), r1(p["sigma1"]["b"]),
            bf(p["msg1"]["w"].T), r1(p["msg1"]["b"]),
            Wh.T, r1(bhd)]


def kernel(x, pos, edge_index, edge_attr, edge_vec, pos_frac, params):
    row = edge_index[0]
    col = edge_index[1]
    h = _dense_call(x, params["lin_in"]["w"].T,
                    params["lin_in"]["b"].reshape(1, -1))
    eav = jnp.concatenate(
        [edge_attr, edge_vec, jnp.zeros((E, 32 - 19), jnp.float32)], axis=1)
    zeros_nd = jnp.zeros((N, D), jnp.float32)
    pf_pad = jnp.zeros((N, D - H - 3), jnp.float32)
    for lp in params["layers"]:
        weights = _layer_weights(lp)
        rowtab = jnp.concatenate([h, pos_frac, pf_pad], axis=1)
        coltab = jnp.concatenate(
            [h, pos_frac, pos, jnp.zeros((N, D - H - 6), jnp.float32)],
            axis=1)
        ge_r, ge_c = _sc_gather(rowtab, coltab, row, col)
        payload = _edge_call(ge_r, ge_c, eav, weights)
        parts = _sc_scatter(payload, col, zeros_nd)
        h, pos = _node_call(parts, h, pos,
                            lp["out0"]["w"].T, lp["out0"]["b"].reshape(1, -1),
                            lp["out1"]["w"].T, lp["out1"]["b"].reshape(1, -1))
    h_fin, pooled = _final_call(h, params["lin_out"]["w"].T,
                                params["lin_out"]["b"].reshape(1, -1))
    return h_fin, pooled


# speedup vs baseline: 1.0075x; 1.0075x over previous
import functools

import jax
import jax.numpy as jnp
import numpy as np
from jax import lax
from jax.experimental import pallas as pl
from jax.experimental.pallas import tpu as pltpu
from jax.experimental.pallas import tpu_sc as plsc

N = 10000
E = 320000
H = 128
D = 144
NC = 2
NS = 16
NW = NC * NS
PER_W = E // NW
C = 80
NCH = PER_W // C
ROWS_PER_TILE = N // NS

EB = 1600
NB = 1000

_INV_SQRT_H = float(1.0 / np.sqrt(float(H)))


def _sigm(v):
    return 0.5 * jnp.tanh(0.5 * v) + 0.5


def _silu(v):
    return v * _sigm(v)


_sc_mesh = plsc.VectorSubcoreMesh(core_axis_name="c", subcore_axis_name="s")


def _gather_body(rowtab, coltab, ridx, cidx, out_r, out_c,
                 idxr_v, idxc_v, bufr, bufc, semr, semc):
    wid = lax.axis_index("c") * NS + lax.axis_index("s")
    base = wid * PER_W

    def body(ck, carry):
        off = base + ck * C
        pltpu.sync_copy(ridx.at[pl.ds(off, C)], idxr_v)
        pltpu.sync_copy(cidx.at[pl.ds(off, C)], idxc_v)
        cp1 = pltpu.async_copy(rowtab.at[idxr_v], bufr, semr)
        cp2 = pltpu.async_copy(coltab.at[idxc_v], bufc, semc)
        cp1.wait()
        cp2.wait()
        pltpu.sync_copy(bufr, out_r.at[pl.ds(off, C)])
        pltpu.sync_copy(bufc, out_c.at[pl.ds(off, C)])
        return carry

    lax.fori_loop(0, NCH, body, 0)


_sc_gather = pl.kernel(
    _gather_body,
    out_type=[jax.ShapeDtypeStruct((E, D), jnp.float32),
              jax.ShapeDtypeStruct((E, D), jnp.float32)],
    mesh=_sc_mesh,
    compiler_params=pltpu.CompilerParams(use_tc_tiling_on_sc=False),
    scratch_types=[
        pltpu.VMEM((C,), jnp.int32),
        pltpu.VMEM((C,), jnp.int32),
        pltpu.VMEM((C, D), jnp.float32),
        pltpu.VMEM((C, D), jnp.float32),
        pltpu.SemaphoreType.DMA,
        pltpu.SemaphoreType.DMA,
    ],
)


def _scatter_body(payload, cidx, zeros, out, idx_v, pay_v, acc):
    cid = lax.axis_index("c")
    sid = lax.axis_index("s")
    wid = cid * NS + sid
    rows0 = sid * ROWS_PER_TILE
    pltpu.sync_copy(zeros.at[pl.ds(rows0, ROWS_PER_TILE)],
                    acc.at[pl.ds(rows0, ROWS_PER_TILE)])
    plsc.subcore_barrier()
    base = wid * PER_W

    def body(ck, carry):
        off = base + ck * C
        pltpu.sync_copy(cidx.at[pl.ds(off, C)], idx_v)
        pltpu.sync_copy(payload.at[pl.ds(off, C)], pay_v)
        pltpu.sync_copy(pay_v, acc.at[idx_v], add=True)
        return carry

    lax.fori_loop(0, NCH, body, 0)
    plsc.subcore_barrier()
    pltpu.sync_copy(acc.at[pl.ds(rows0, ROWS_PER_TILE)],
                    out.at[pl.ds(cid * N + rows0, ROWS_PER_TILE)])


_sc_scatter = pl.kernel(
    _scatter_body,
    out_type=jax.ShapeDtypeStruct((2 * N, D), jnp.float32),
    mesh=_sc_mesh,
    compiler_params=pltpu.CompilerParams(use_tc_tiling_on_sc=False),
    scratch_types=[
        pltpu.VMEM((C,), jnp.int32),
        pltpu.VMEM((C, D), jnp.float32),
        pltpu.VMEM_SHARED((N, D), jnp.float32),
    ],
)


def _edge_body(ger, gec, eav, wr, br, wc, bc, wle, ble, we2, be2, ws8,
               s1w, s1b, m1w, m1b, wh, bh, out):
    hr = ger[:, :H].astype(jnp.bfloat16)
    hc = gec[:, :H].astype(jnp.bfloat16)
    zr = jnp.dot(hr, wr[...], preferred_element_type=jnp.float32) + br[...]
    zc = jnp.dot(hc, wc[...], preferred_element_type=jnp.float32) + bc[...]
    ea = eav[:, :16]
    e = jnp.dot(ea, wle[...], preferred_element_type=jnp.float32) + ble[...]
    ez = jnp.dot(e, we2[...], preferred_element_type=jnp.float32) + be2[...]
    sig_in = ez[:, :H] + zc[:, :H]
    kk = _silu(jnp.dot(_silu(sig_in).astype(jnp.bfloat16), s1w[...],
                       preferred_element_type=jnp.float32) + s1b[...])
    q = zr[:, :H]
    w = _sigm(q * kk * _INV_SQRT_H)
    m_in = ez[:, H:] + zr[:, H:2 * H] + zc[:, H:2 * H]
    msg = _silu(jnp.dot(_silu(m_in).astype(jnp.bfloat16), m1w[...],
                        preferred_element_type=jnp.float32) + m1b[...])
    m_ij = w * msg

    ev = eav[:, 16:19]
    pr = ger[:, H:H + 3]
    pc = gec[:, H:H + 3]
    posc = gec[:, H + 3:H + 6]
    s8 = jnp.concatenate([
        jnp.sum(ev * pr, axis=1, keepdims=True),
        jnp.sum(ev * pc, axis=1, keepdims=True),
        jnp.sum(pr * pc, axis=1, keepdims=True),
        jnp.sqrt(jnp.sum(ev * ev, axis=1, keepdims=True)),
        jnp.sqrt(jnp.sum(pr * pr, axis=1, keepdims=True)),
        jnp.sqrt(jnp.sum(pc * pc, axis=1, keepdims=True)),
        jnp.zeros((ev.shape[0], 2), jnp.float32)], axis=1)
    sz = jnp.dot(s8, ws8[...], preferred_element_type=jnp.float32)
    a_in = sz[:, :H] + zr[:, 2 * H:3 * H] + zc[:, 2 * H:3 * H]
    b_in = sz[:, H:2 * H] + zr[:, 3 * H:4 * H] + zc[:, 3 * H:4 * H]
    g_in = sz[:, 2 * H:] + zr[:, 4 * H:] + zc[:, 4 * H:]
    cat3 = jnp.concatenate([_silu(a_in), _silu(b_in), _silu(g_in)], axis=1)
    abg = jnp.dot(cat3, wh[...], preferred_element_type=jnp.float32) + bh[...]
    a = abg[:, 0:3]
    b = _sigm(abg[:, 3:6])
    g = jnp.tanh(abg[:, 6:9])
    pv = a * (ev * g) + b * posc
    nb = ev.shape[0]
    out[...] = jnp.concatenate(
        [m_ij, pv, jnp.ones((nb, 1), jnp.float32),
         jnp.zeros((nb, D - H - 4), jnp.float32)], axis=1)


def _edge_call(ger, gec, eav, weights):
    full = lambda shape: pl.BlockSpec(shape, lambda i: (0, 0))
    wspecs = [full(w.shape) for w in weights]
    return pl.pallas_call(
        _edge_body,
        grid=(E // EB,),
        in_specs=[pl.BlockSpec((EB, D), lambda i: (i, 0)),
                  pl.BlockSpec((EB, D), lambda i: (i, 0)),
                  pl.BlockSpec((EB, 32), lambda i: (i, 0))] + wspecs,
        out_specs=pl.BlockSpec((EB, D), lambda i: (i, 0)),
        out_shape=jax.ShapeDtypeStruct((E, D), jnp.float32),
    )(ger, gec, eav, *weights)


def _node_body(p0, p1, h, pos, w0, b0, w1, b1, hout, posout):
    acc = p0[...] + p1[...]
    cnt = jnp.maximum(acc[:, H + 3:H + 4], 1.0)
    agg = acc[:, :H] / cnt
    dpos = acc[:, H:H + 3] / cnt
    hcat = jnp.concatenate([agg, h[...]], axis=1)
    u = _silu(jnp.dot(hcat, w0[...], preferred_element_type=jnp.float32)
              + b0[...])
    hout[...] = jnp.dot(u, w1[...], preferred_element_type=jnp.float32) \
        + b1[...] + h[...]
    posout[...] = pos[...] + dpos


def _node_call(parts, h, pos, w0, b0, w1, b1):
    nblk = N // NB
    full = lambda shape: pl.BlockSpec(shape, lambda i: (0, 0))
    return pl.pallas_call(
        _node_body,
        grid=(nblk,),
        in_specs=[pl.BlockSpec((NB, D), lambda i: (i, 0)),
                  pl.BlockSpec((NB, D), lambda i: (i + nblk, 0)),
                  pl.BlockSpec((NB, H), lambda i: (i, 0)),
                  pl.BlockSpec((NB, 3), lambda i: (i, 0)),
                  full(w0.shape), full(b0.shape),
                  full(w1.shape), full(b1.shape)],
        out_specs=[pl.BlockSpec((NB, H), lambda i: (i, 0)),
                   pl.BlockSpec((NB, 3), lambda i: (i, 0))],
        out_shape=[jax.ShapeDtypeStruct((N, H), jnp.float32),
                   jax.ShapeDtypeStruct((N, 3), jnp.float32)],
    )(parts, parts, h, pos, w0, b0, w1, b1)


def _dense_body(x, w, b, out):
    out[...] = jnp.dot(x[...], w[...],
                       preferred_element_type=jnp.float32) + b[...]


def _dense_call(x, w, b):
    n, di = x.shape
    do = w.shape[1]
    full = lambda shape: pl.BlockSpec(shape, lambda i: (0, 0))
    return pl.pallas_call(
        _dense_body,
        grid=(n // NB,),
        in_specs=[pl.BlockSpec((NB, di), lambda i: (i, 0)),
                  full(w.shape), full(b.shape)],
        out_specs=pl.BlockSpec((NB, do), lambda i: (i, 0)),
        out_shape=jax.ShapeDtypeStruct((n, do), jnp.float32),
    )(x, w, b)


def _final_body(h, w, b, hout, pooled):
    i = pl.program_id(0)
    hf = jnp.dot(h[...], w[...], preferred_element_type=jnp.float32) + b[...]
    hout[...] = hf
    s = jnp.sum(hf, axis=0, keepdims=True)

    @pl.when(i == 0)
    def _():
        pooled[...] = s

    @pl.when(i > 0)
    def _():
        pooled[...] += s

    @pl.when(i == (N // NB) - 1)
    def _():
        pooled[...] = pooled[...] * (1.0 / N)


def _final_call(h, w, b):
    full = lambda shape: pl.BlockSpec(shape, lambda i: (0, 0))
    return pl.pallas_call(
        _final_body,
        grid=(N // NB,),
        in_specs=[pl.BlockSpec((NB, H), lambda i: (i, 0)),
                  full(w.shape), full(b.shape)],
        out_specs=[pl.BlockSpec((NB, H), lambda i: (i, 0)),
                   pl.BlockSpec((1, H), lambda i: (0, 0))],
        out_shape=[jax.ShapeDtypeStruct((N, H), jnp.float32),
                   jax.ShapeDtypeStruct((1, H), jnp.float32)],
    )(h, w, b)


def _layer_weights(p):
    Qw, Qb = p["LnQ"]["w"], p["LnQ"]["b"]
    Kw, Kb = p["LnK"]["w"], p["LnK"]["b"]
    Vw, Vb = p["LnV"]["w"], p["LnV"]["b"]
    s0w, s0b = p["sigma0"]["w"], p["sigma0"]["b"]
    s0we, s0wk = s0w[:, :16], s0w[:, 16:]
    m0w, m0b = p["msg0"]["w"], p["msg0"]["b"]
    m0we, m0wr, m0wc = m0w[:, :16], m0w[:, 16:16 + H], m0w[:, 16 + H:]
    a0w, a0b = p["alpha0"]["w"], p["alpha0"]["b"]
    b0w, b0b = p["beta0"]["w"], p["beta0"]["b"]
    g0w, g0b = p["gate0"]["w"], p["gate0"]["b"]
    z = jnp.zeros((3 * H,), jnp.float32)
    W_r = jnp.concatenate([Qw, m0wr @ Vw, a0w[:, 6 + H:], b0w[:, 6 + H:],
                           g0w[:, 6 + H:]], axis=0)
    b_r = jnp.concatenate([Qb, m0wr @ Vb, z], axis=0)
    W_c = jnp.concatenate([s0wk @ Kw, m0wc @ Vw, a0w[:, 6:6 + H],
                           b0w[:, 6:6 + H], g0w[:, 6:6 + H]], axis=0)
    b_c = jnp.concatenate([s0wk @ Kb, m0wc @ Vb, a0b, b0b, g0b], axis=0)
    W_e2 = jnp.concatenate([s0we, m0we], axis=0)
    b_e2 = jnp.concatenate([s0b, m0b], axis=0)
    ws6 = jnp.concatenate([a0w[:, :6], b0w[:, :6], g0w[:, :6]], axis=0)
    W_s8 = jnp.concatenate([ws6, jnp.zeros((3 * H, 2), jnp.float32)], axis=1)
    zh = jnp.zeros((3, H), jnp.float32)
    Wh = jnp.concatenate([
        jnp.concatenate([p["alpha1"]["w"], zh, zh], axis=1),
        jnp.concatenate([zh, p["beta1"]["w"], zh], axis=1),
        jnp.concatenate([zh, zh, p["gate1"]["w"]], axis=1)], axis=0)
    bhd = jnp.concatenate([p["alpha1"]["b"], p["beta1"]["b"],
                           p["gate1"]["b"]], axis=0)
    r1 = lambda v: v.reshape(1, -1)
    bf = lambda v: v.astype(jnp.bfloat16)
    return [bf(W_r.T), r1(b_r), bf(W_c.T), r1(b_c),
            p["LnE"]["w"].T, r1(p["LnE"]["b"]),
            W_e2.T, r1(b_e2), W_s8.T,
            bf(p["sigma1"]["w"].T), r1(p["sigma1"]["b"]),
            bf(p["msg1"]["w"].T), r1(p["msg1"]["b"]),
            Wh.T, r1(bhd)]


def kernel(x, pos, edge_index, edge_attr, edge_vec, pos_frac, params):
    row = edge_index[0]
    col = edge_index[1]
    h = _dense_call(x, params["lin_in"]["w"].T,
                    params["lin_in"]["b"].reshape(1, -1))
    eav = jnp.concatenate(
        [edge_attr, edge_vec, jnp.zeros((E, 32 - 19), jnp.float32)], axis=1)
    zeros_nd = jnp.zeros((N, D), jnp.float32)
    pf_pad = jnp.zeros((N, D - H - 3), jnp.float32)
    for lp in params["layers"]:
        weights = _layer_weights(lp)
        rowtab = jnp.concatenate([h, pos_frac, pf_pad], axis=1)
        coltab = jnp.concatenate(
            [h, pos_frac, pos, jnp.zeros((N, D - H - 6), jnp.float32)],
            axis=1)
        ge_r, ge_c = _sc_gather(rowtab, coltab, row, col)
        payload = _edge_call(ge_r, ge_c, eav, weights)
        parts = _sc_scatter(payload, col, zeros_nd)
        h, pos = _node_call(parts, h, pos,
                            lp["out0"]["w"].T, lp["out0"]["b"].reshape(1, -1),
                            lp["out1"]["w"].T, lp["out1"]["b"].reshape(1, -1))
    h_fin, pooled = _final_call(h, params["lin_out"]["w"].T,
                                params["lin_out"]["b"].reshape(1, -1))
    return h_fin, pooled


# speedup vs baseline: 1.2925x; 1.2829x over previous
import jax
import jax.numpy as jnp
import numpy as np
from jax import lax
from jax.experimental import pallas as pl
from jax.experimental.pallas import tpu as pltpu
from jax.experimental.pallas import tpu_sc as plsc

N = 10000
E = 320000
H = 128
SW = 16
S8 = 8
D = 144
NC = 2
NS = 16
NW = NC * NS
PER_W = E // NW
C = 80
NCH = PER_W // C
ROWS_PER_TILE = N // NS
RPT8 = 624

EB = 1600
NB = 1000

_INV_SQRT_H = float(1.0 / np.sqrt(float(H)))


def _sigm(v):
    return 0.5 * jnp.tanh(0.5 * v) + 0.5


def _silu(v):
    return v * _sigm(v)


_sc_mesh = plsc.VectorSubcoreMesh(core_axis_name="c", subcore_axis_name="s")


def _gather_h_body(htab, ridx, cidx, out_r, out_c,
                   idxr_v, idxc_v, bufr, bufc, semr, semc):
    wid = lax.axis_index("c") * NS + lax.axis_index("s")
    base = wid * PER_W

    def body(ck, carry):
        off = base + ck * C
        pltpu.sync_copy(ridx.at[pl.ds(off, C)], idxr_v)
        pltpu.sync_copy(cidx.at[pl.ds(off, C)], idxc_v)
        cp1 = pltpu.async_copy(htab.at[idxr_v], bufr, semr)
        cp2 = pltpu.async_copy(htab.at[idxc_v], bufc, semc)
        cp1.wait()
        cp2.wait()
        pltpu.sync_copy(bufr, out_r.at[pl.ds(off, C)])
        pltpu.sync_copy(bufc, out_c.at[pl.ds(off, C)])
        return carry

    lax.fori_loop(0, NCH, body, 0)


_sc_gather_h = pl.kernel(
    _gather_h_body,
    out_type=[jax.ShapeDtypeStruct((E, H), jnp.float32),
              jax.ShapeDtypeStruct((E, H), jnp.float32)],
    mesh=_sc_mesh,
    scratch_types=[
        pltpu.VMEM((C,), jnp.int32),
        pltpu.VMEM((C,), jnp.int32),
        pltpu.VMEM((C, H), jnp.float32),
        pltpu.VMEM((C, H), jnp.float32),
        pltpu.SemaphoreType.DMA,
        pltpu.SemaphoreType.DMA,
    ],
)


def _gather_pf_body(pftab, ridx, cidx, out_r, out_c,
                    idxr_v, idxc_v, bufr, bufc, semr, semc):
    wid = lax.axis_index("c") * NS + lax.axis_index("s")
    base = wid * PER_W

    def body(ck, carry):
        off = base + ck * C
        pltpu.sync_copy(ridx.at[pl.ds(off, C)], idxr_v)
        pltpu.sync_copy(cidx.at[pl.ds(off, C)], idxc_v)
        cp1 = pltpu.async_copy(pftab.at[idxr_v], bufr, semr)
        cp2 = pltpu.async_copy(pftab.at[idxc_v], bufc, semc)
        cp1.wait()
        cp2.wait()
        pltpu.sync_copy(bufr, out_r.at[pl.ds(off, C)])
        pltpu.sync_copy(bufc, out_c.at[pl.ds(off, C)])
        return carry

    lax.fori_loop(0, NCH, body, 0)


_sc_gather_pf = pl.kernel(
    _gather_pf_body,
    out_type=[jax.ShapeDtypeStruct((E, SW), jnp.float32),
              jax.ShapeDtypeStruct((E, SW), jnp.float32)],
    mesh=_sc_mesh,
    compiler_params=pltpu.CompilerParams(use_tc_tiling_on_sc=False),
    scratch_types=[
        pltpu.VMEM((C,), jnp.int32),
        pltpu.VMEM((C,), jnp.int32),
        pltpu.VMEM((C, SW), jnp.float32),
        pltpu.VMEM((C, SW), jnp.float32),
        pltpu.SemaphoreType.DMA,
        pltpu.SemaphoreType.DMA,
    ],
)


def _scatter_body(payload, cidx, zeros, out, idx_v, pay_v, acc):
    cid = lax.axis_index("c")
    sid = lax.axis_index("s")
    wid = cid * NS + sid
    rows0 = sid * ROWS_PER_TILE
    pltpu.sync_copy(zeros.at[pl.ds(rows0, ROWS_PER_TILE)],
                    acc.at[pl.ds(rows0, ROWS_PER_TILE)])
    plsc.subcore_barrier()
    base = wid * PER_W

    def body(ck, carry):
        off = base + ck * C
        pltpu.sync_copy(cidx.at[pl.ds(off, C)], idx_v)
        pltpu.sync_copy(payload.at[pl.ds(off, C)], pay_v)
        pltpu.sync_copy(pay_v, acc.at[idx_v], add=True)
        return carry

    lax.fori_loop(0, NCH, body, 0)
    plsc.subcore_barrier()
    pltpu.sync_copy(acc.at[pl.ds(rows0, ROWS_PER_TILE)],
                    out.at[pl.ds(cid * N + rows0, ROWS_PER_TILE)])


_sc_scatter = pl.kernel(
    _scatter_body,
    out_type=jax.ShapeDtypeStruct((2 * N, D), jnp.float32),
    mesh=_sc_mesh,
    compiler_params=pltpu.CompilerParams(use_tc_tiling_on_sc=False),
    scratch_types=[
        pltpu.VMEM((C,), jnp.int32),
        pltpu.VMEM((C, D), jnp.float32),
        pltpu.VMEM_SHARED((N, D), jnp.float32),
    ],
)


def _prep_body(smr, smc, eav, out):
    pr = smr[:, 0:3]
    pc = smc[:, 0:3]
    ev = eav[:, 16:19]
    nb = ev.shape[0]
    out[...] = jnp.concatenate([
        jnp.sum(ev * pr, axis=1, keepdims=True),
        jnp.sum(ev * pc, axis=1, keepdims=True),
        jnp.sum(pr * pc, axis=1, keepdims=True),
        jnp.sqrt(jnp.sum(ev * ev, axis=1, keepdims=True)),
        jnp.sqrt(jnp.sum(pr * pr, axis=1, keepdims=True)),
        jnp.sqrt(jnp.sum(pc * pc, axis=1, keepdims=True)),
        jnp.zeros((nb, 2), jnp.float32)], axis=1)


def _prep_call(smr, smc, eav):
    return pl.pallas_call(
        _prep_body,
        grid=(E // EB,),
        in_specs=[pl.BlockSpec((EB, SW), lambda i: (i, 0)),
                  pl.BlockSpec((EB, SW), lambda i: (i, 0)),
                  pl.BlockSpec((EB, 32), lambda i: (i, 0))],
        out_specs=pl.BlockSpec((EB, S8), lambda i: (i, 0)),
        out_shape=jax.ShapeDtypeStruct((E, S8), jnp.float32),
    )(smr, smc, eav)


def _edge_body(ger, gec, eav, s8e, wr, br, wc, bc, wle, ble, we2, be2, ws8,
               s1w, s1b, m1w, m1b, wh, bh, out):
    hr = ger[...].astype(jnp.bfloat16)
    hc = gec[...].astype(jnp.bfloat16)
    zr = jnp.dot(hr, wr[...], preferred_element_type=jnp.float32) + br[...]
    zc = jnp.dot(hc, wc[...], preferred_element_type=jnp.float32) + bc[...]
    ea = eav[:, :16]
    e = jnp.dot(ea, wle[...], preferred_element_type=jnp.float32) + ble[...]
    ez = jnp.dot(e, we2[...], preferred_element_type=jnp.float32) + be2[...]
    sig_in = ez[:, :H] + zc[:, :H]
    kk = _silu(jnp.dot(_silu(sig_in).astype(jnp.bfloat16), s1w[...],
                       preferred_element_type=jnp.float32) + s1b[...])
    q = zr[:, :H]
    w = _sigm(q * kk * _INV_SQRT_H)
    m_in = ez[:, H:] + zr[:, H:2 * H] + zc[:, H:2 * H]
    msg = _silu(jnp.dot(_silu(m_in).astype(jnp.bfloat16), m1w[...],
                        preferred_element_type=jnp.float32) + m1b[...])
    m_ij = w * msg

    sz = jnp.dot(s8e[...], ws8[...], preferred_element_type=jnp.float32)
    a_in = sz[:, :H] + zr[:, 2 * H:3 * H] + zc[:, 2 * H:3 * H]
    b_in = sz[:, H:2 * H] + zr[:, 3 * H:4 * H] + zc[:, 3 * H:4 * H]
    g_in = sz[:, 2 * H:] + zr[:, 4 * H:] + zc[:, 4 * H:]
    cat3 = jnp.concatenate([_silu(a_in), _silu(b_in), _silu(g_in)], axis=1)
    abg = jnp.dot(cat3, wh[...], preferred_element_type=jnp.float32) + bh[...]
    ev = eav[:, 16:19]
    a = abg[:, 0:3]
    b = _sigm(abg[:, 3:6])
    g = jnp.tanh(abg[:, 6:9])
    apart = a * (ev * g)
    nb = ev.shape[0]
    out[...] = jnp.concatenate(
        [m_ij, apart, jnp.ones((nb, 1), jnp.float32), b,
         jnp.zeros((nb, D - H - 7), jnp.float32)], axis=1)


def _edge_call(ger, gec, eav, s8e, weights):
    full = lambda shape: pl.BlockSpec(shape, lambda i: (0, 0))
    wspecs = [full(w.shape) for w in weights]
    return pl.pallas_call(
        _edge_body,
        grid=(E // EB,),
        in_specs=[pl.BlockSpec((EB, H), lambda i: (i, 0)),
                  pl.BlockSpec((EB, H), lambda i: (i, 0)),
                  pl.BlockSpec((EB, 32), lambda i: (i, 0)),
                  pl.BlockSpec((EB, S8), lambda i: (i, 0))] + wspecs,
        out_specs=pl.BlockSpec((EB, D), lambda i: (i, 0)),
        out_shape=jax.ShapeDtypeStruct((E, D), jnp.float32),
    )(ger, gec, eav, s8e, *weights)


def _node_body(p0, p1, h, pos, w0, b0, w1, b1, hout, posout):
    acc = p0[...] + p1[...]
    cnt = jnp.maximum(acc[:, H + 3:H + 4], 1.0)
    agg = acc[:, :H] / cnt
    dpos = (acc[:, H:H + 3] + acc[:, H + 4:H + 7] * pos[...]) / cnt
    hcat = jnp.concatenate([agg, h[...]], axis=1)
    u = _silu(jnp.dot(hcat, w0[...], preferred_element_type=jnp.float32)
              + b0[...])
    hout[...] = jnp.dot(u, w1[...], preferred_element_type=jnp.float32) \
        + b1[...] + h[...]
    posout[...] = pos[...] + dpos


def _node_call(parts, h, pos, w0, b0, w1, b1):
    nblk = N // NB
    full = lambda shape: pl.BlockSpec(shape, lambda i: (0, 0))
    return pl.pallas_call(
        _node_body,
        grid=(nblk,),
        in_specs=[pl.BlockSpec((NB, D), lambda i: (i, 0)),
                  pl.BlockSpec((NB, D), lambda i: (i + nblk, 0)),
                  pl.BlockSpec((NB, H), lambda i: (i, 0)),
                  pl.BlockSpec((NB, 3), lambda i: (i, 0)),
                  full(w0.shape), full(b0.shape),
                  full(w1.shape), full(b1.shape)],
        out_specs=[pl.BlockSpec((NB, H), lambda i: (i, 0)),
                   pl.BlockSpec((NB, 3), lambda i: (i, 0))],
        out_shape=[jax.ShapeDtypeStruct((N, H), jnp.float32),
                   jax.ShapeDtypeStruct((N, 3), jnp.float32)],
    )(parts, parts, h, pos, w0, b0, w1, b1)


def _dense_body(x, w, b, out):
    out[...] = jnp.dot(x[...], w[...],
                       preferred_element_type=jnp.float32) + b[...]


def _dense_call(x, w, b):
    n, di = x.shape
    do = w.shape[1]
    full = lambda shape: pl.BlockSpec(shape, lambda i: (0, 0))
    return pl.pallas_call(
        _dense_body,
        grid=(n // NB,),
        in_specs=[pl.BlockSpec((NB, di), lambda i: (i, 0)),
                  full(w.shape), full(b.shape)],
        out_specs=pl.BlockSpec((NB, do), lambda i: (i, 0)),
        out_shape=jax.ShapeDtypeStruct((n, do), jnp.float32),
    )(x, w, b)


def _final_body(h, w, b, hout, pooled):
    i = pl.program_id(0)
    hf = jnp.dot(h[...], w[...], preferred_element_type=jnp.float32) + b[...]
    hout[...] = hf
    s = jnp.sum(hf, axis=0, keepdims=True)

    @pl.when(i == 0)
    def _():
        pooled[...] = s

    @pl.when(i > 0)
    def _():
        pooled[...] += s

    @pl.when(i == (N // NB) - 1)
    def _():
        pooled[...] = pooled[...] * (1.0 / N)


def _final_call(h, w, b):
    full = lambda shape: pl.BlockSpec(shape, lambda i: (0, 0))
    return pl.pallas_call(
        _final_body,
        grid=(N // NB,),
        in_specs=[pl.BlockSpec((NB, H), lambda i: (i, 0)),
                  full(w.shape), full(b.shape)],
        out_specs=[pl.BlockSpec((NB, H), lambda i: (i, 0)),
                   pl.BlockSpec((1, H), lambda i: (0, 0))],
        out_shape=[jax.ShapeDtypeStruct((N, H), jnp.float32),
                   jax.ShapeDtypeStruct((1, H), jnp.float32)],
    )(h, w, b)


def _layer_weights(p):
    Qw, Qb = p["LnQ"]["w"], p["LnQ"]["b"]
    Kw, Kb = p["LnK"]["w"], p["LnK"]["b"]
    Vw, Vb = p["LnV"]["w"], p["LnV"]["b"]
    s0w, s0b = p["sigma0"]["w"], p["sigma0"]["b"]
    s0we, s0wk = s0w[:, :16], s0w[:, 16:]
    m0w, m0b = p["msg0"]["w"], p["msg0"]["b"]
    m0we, m0wr, m0wc = m0w[:, :16], m0w[:, 16:16 + H], m0w[:, 16 + H:]
    a0w, a0b = p["alpha0"]["w"], p["alpha0"]["b"]
    b0w, b0b = p["beta0"]["w"], p["beta0"]["b"]
    g0w, g0b = p["gate0"]["w"], p["gate0"]["b"]
    z = jnp.zeros((3 * H,), jnp.float32)
    W_r = jnp.concatenate([Qw, m0wr @ Vw, a0w[:, 6 + H:], b0w[:, 6 + H:],
                           g0w[:, 6 + H:]], axis=0)
    b_r = jnp.concatenate([Qb, m0wr @ Vb, z], axis=0)
    W_c = jnp.concatenate([s0wk @ Kw, m0wc @ Vw, a0w[:, 6:6 + H],
                           b0w[:, 6:6 + H], g0w[:, 6:6 + H]], axis=0)
    b_c = jnp.concatenate([s0wk @ Kb, m0wc @ Vb, a0b, b0b, g0b], axis=0)
    W_e2 = jnp.concatenate([s0we, m0we], axis=0)
    b_e2 = jnp.concatenate([s0b, m0b], axis=0)
    ws6 = jnp.concatenate([a0w[:, :6], b0w[:, :6], g0w[:, :6]], axis=0)
    W_s8 = jnp.concatenate([ws6, jnp.zeros((3 * H, 2), jnp.float32)], axis=1)
    zh = jnp.zeros((3, H), jnp.float32)
    Wh = jnp.concatenate([
        jnp.concatenate([p["alpha1"]["w"], zh, zh], axis=1),
        jnp.concatenate([zh, p["beta1"]["w"], zh], axis=1),
        jnp.concatenate([zh, zh, p["gate1"]["w"]], axis=1)], axis=0)
    bhd = jnp.concatenate([p["alpha1"]["b"], p["beta1"]["b"],
                           p["gate1"]["b"]], axis=0)
    r1 = lambda v: v.reshape(1, -1)
    bf = lambda v: v.astype(jnp.bfloat16)
    return [bf(W_r.T), r1(b_r), bf(W_c.T), r1(b_c),
            p["LnE"]["w"].T, r1(p["LnE"]["b"]),
            W_e2.T, r1(b_e2), W_s8.T,
            bf(p["sigma1"]["w"].T), r1(p["sigma1"]["b"]),
            bf(p["msg1"]["w"].T), r1(p["msg1"]["b"]),
            Wh.T, r1(bhd)]


def kernel(x, pos, edge_index, edge_attr, edge_vec, pos_frac, params):
    row = edge_index[0]
    col = edge_index[1]
    h = _dense_call(x, params["lin_in"]["w"].T,
                    params["lin_in"]["b"].reshape(1, -1))
    eav = jnp.concatenate(
        [edge_attr, edge_vec, jnp.zeros((E, 32 - 19), jnp.float32)], axis=1)
    pftab = jnp.concatenate(
        [pos_frac, jnp.zeros((N, SW - 3), jnp.float32)], axis=1)
    smr, smc = _sc_gather_pf(pftab, row, col)
    s8e = _prep_call(smr, smc, eav)
    zerosd = jnp.zeros((N, D), jnp.float32)
    for lp in params["layers"]:
        weights = _layer_weights(lp)
        ge_r, ge_c = _sc_gather_h(h, row, col)
        payload = _edge_call(ge_r, ge_c, eav, s8e, weights)
        parts = _sc_scatter(payload, col, zerosd)
        h, pos = _node_call(parts, h, pos,
                            lp["out0"]["w"].T, lp["out0"]["b"].reshape(1, -1),
                            lp["out1"]["w"].T, lp["out1"]["b"].reshape(1, -1))
    h_fin, pooled = _final_call(h, params["lin_out"]["w"].T,
                                params["lin_out"]["b"].reshape(1, -1))
    return h_fin, pooled


# speedup vs baseline: 1.3272x; 1.0268x over previous
import jax
import jax.numpy as jnp
import numpy as np
from jax import lax
from jax.experimental import pallas as pl
from jax.experimental.pallas import tpu as pltpu
from jax.experimental.pallas import tpu_sc as plsc

N = 10000
E = 320000
H = 128
SW = 16
S8 = 8
D = 144
NC = 2
NS = 16
NW = NC * NS
PER_W = E // NW
C = 80
NCH = PER_W // C
ROWS_PER_TILE = N // NS
RPT8 = 624

EB = 1600
NB = 1000

_INV_SQRT_H = float(1.0 / np.sqrt(float(H)))


def _sigm(v):
    return 0.5 * jnp.tanh(0.5 * v) + 0.5


def _silu(v):
    return v * _sigm(v)


_sc_mesh = plsc.VectorSubcoreMesh(core_axis_name="c", subcore_axis_name="s")


def _gather_h_body(htab, ridx, cidx, out_r, out_c,
                   idxr_v, idxc_v, bufr, bufc, semr, semc):
    wid = lax.axis_index("c") * NS + lax.axis_index("s")
    base = wid * PER_W

    def body(ck, carry):
        off = base + ck * C
        pltpu.sync_copy(ridx.at[pl.ds(off, C)], idxr_v)
        pltpu.sync_copy(cidx.at[pl.ds(off, C)], idxc_v)
        cp1 = pltpu.async_copy(htab.at[idxr_v], bufr, semr)
        cp2 = pltpu.async_copy(htab.at[idxc_v], bufc, semc)
        cp1.wait()
        cp2.wait()
        pltpu.sync_copy(bufr, out_r.at[pl.ds(off, C)])
        pltpu.sync_copy(bufc, out_c.at[pl.ds(off, C)])
        return carry

    lax.fori_loop(0, NCH, body, 0)


_sc_gather_h = pl.kernel(
    _gather_h_body,
    out_type=[jax.ShapeDtypeStruct((E, H), jnp.float32),
              jax.ShapeDtypeStruct((E, H), jnp.float32)],
    mesh=_sc_mesh,
    scratch_types=[
        pltpu.VMEM((C,), jnp.int32),
        pltpu.VMEM((C,), jnp.int32),
        pltpu.VMEM((C, H), jnp.float32),
        pltpu.VMEM((C, H), jnp.float32),
        pltpu.SemaphoreType.DMA,
        pltpu.SemaphoreType.DMA,
    ],
)


def _gather_pf_body(pftab, ridx, cidx, out_r, out_c,
                    idxr_v, idxc_v, bufr, bufc, semr, semc):
    wid = lax.axis_index("c") * NS + lax.axis_index("s")
    base = wid * PER_W

    def body(ck, carry):
        off = base + ck * C
        pltpu.sync_copy(ridx.at[pl.ds(off, C)], idxr_v)
        pltpu.sync_copy(cidx.at[pl.ds(off, C)], idxc_v)
        cp1 = pltpu.async_copy(pftab.at[idxr_v], bufr, semr)
        cp2 = pltpu.async_copy(pftab.at[idxc_v], bufc, semc)
        cp1.wait()
        cp2.wait()
        pltpu.sync_copy(bufr, out_r.at[pl.ds(off, C)])
        pltpu.sync_copy(bufc, out_c.at[pl.ds(off, C)])
        return carry

    lax.fori_loop(0, NCH, body, 0)


_sc_gather_pf = pl.kernel(
    _gather_pf_body,
    out_type=[jax.ShapeDtypeStruct((E, SW), jnp.float32),
              jax.ShapeDtypeStruct((E, SW), jnp.float32)],
    mesh=_sc_mesh,
    compiler_params=pltpu.CompilerParams(use_tc_tiling_on_sc=False),
    scratch_types=[
        pltpu.VMEM((C,), jnp.int32),
        pltpu.VMEM((C,), jnp.int32),
        pltpu.VMEM((C, SW), jnp.float32),
        pltpu.VMEM((C, SW), jnp.float32),
        pltpu.SemaphoreType.DMA,
        pltpu.SemaphoreType.DMA,
    ],
)


def _scatter_body(payload, cidx, zeros, out, idx_v, pay_v, acc):
    cid = lax.axis_index("c")
    sid = lax.axis_index("s")
    wid = cid * NS + sid
    rows0 = sid * ROWS_PER_TILE
    pltpu.sync_copy(zeros.at[pl.ds(rows0, ROWS_PER_TILE)],
                    acc.at[pl.ds(rows0, ROWS_PER_TILE)])
    plsc.subcore_barrier()
    base = wid * PER_W

    def body(ck, carry):
        off = base + ck * C
        pltpu.sync_copy(cidx.at[pl.ds(off, C)], idx_v)
        pltpu.sync_copy(payload.at[pl.ds(off, C)], pay_v)
        pltpu.sync_copy(pay_v, acc.at[idx_v], add=True)
        return carry

    lax.fori_loop(0, NCH, body, 0)
    plsc.subcore_barrier()
    pltpu.sync_copy(acc.at[pl.ds(rows0, ROWS_PER_TILE)],
                    out.at[pl.ds(cid * N + rows0, ROWS_PER_TILE)])


_sc_scatter = pl.kernel(
    _scatter_body,
    out_type=jax.ShapeDtypeStruct((2 * N, D), jnp.float32),
    mesh=_sc_mesh,
    compiler_params=pltpu.CompilerParams(use_tc_tiling_on_sc=False),
    scratch_types=[
        pltpu.VMEM((C,), jnp.int32),
        pltpu.VMEM((C, D), jnp.float32),
        pltpu.VMEM_SHARED((N, D), jnp.float32),
    ],
)


def _prep_body(smr, smc, eav, sel1, sel2, out):
    pr = smr[...]
    pc = smc[...]
    ev = eav[:, 16:32]
    prods = jnp.concatenate(
        [ev * pr, ev * pc, pr * pc, ev * ev, pr * pr, pc * pc], axis=1)
    s_lin = jnp.dot(prods, sel1[...], preferred_element_type=jnp.float32)
    s_sq = jnp.sqrt(jnp.dot(prods, sel2[...],
                            preferred_element_type=jnp.float32))
    out[...] = s_lin + s_sq


def _prep_call(smr, smc, eav, sel1, sel2):
    full = lambda shape: pl.BlockSpec(shape, lambda i: (0, 0))
    return pl.pallas_call(
        _prep_body,
        grid=(E // EB,),
        in_specs=[pl.BlockSpec((EB, SW), lambda i: (i, 0)),
                  pl.BlockSpec((EB, SW), lambda i: (i, 0)),
                  pl.BlockSpec((EB, 32), lambda i: (i, 0)),
                  full(sel1.shape), full(sel2.shape)],
        out_specs=pl.BlockSpec((EB, S8), lambda i: (i, 0)),
        out_shape=jax.ShapeDtypeStruct((E, S8), jnp.float32),
    )(smr, smc, eav, sel1, sel2)


def _edge_body(ger, gec, eav, s8e, wr, br, wc, bc, wle, ble, we2, be2, ws8,
               s1w, s1b, m1w, m1b, wh, bh, out):
    hr = ger[...].astype(jnp.bfloat16)
    hc = gec[...].astype(jnp.bfloat16)
    zr = jnp.dot(hr, wr[...], preferred_element_type=jnp.float32) + br[...]
    zc = jnp.dot(hc, wc[...], preferred_element_type=jnp.float32) + bc[...]
    ea = eav[:, :16]
    e = jnp.dot(ea, wle[...], preferred_element_type=jnp.float32) + ble[...]
    ez = jnp.dot(e, we2[...], preferred_element_type=jnp.float32) + be2[...]
    sig_in = ez[:, :H] + zc[:, :H]
    kk = _silu(jnp.dot(_silu(sig_in).astype(jnp.bfloat16), s1w[...],
                       preferred_element_type=jnp.float32) + s1b[...])
    q = zr[:, :H]
    w = _sigm(q * kk * _INV_SQRT_H)
    m_in = ez[:, H:] + zr[:, H:2 * H] + zc[:, H:2 * H]
    msg = _silu(jnp.dot(_silu(m_in).astype(jnp.bfloat16), m1w[...],
                        preferred_element_type=jnp.float32) + m1b[...])
    m_ij = w * msg

    sz = jnp.dot(s8e[...], ws8[...], preferred_element_type=jnp.float32)
    a_in = sz[:, :H] + zr[:, 2 * H:3 * H] + zc[:, 2 * H:3 * H]
    b_in = sz[:, H:2 * H] + zr[:, 3 * H:4 * H] + zc[:, 3 * H:4 * H]
    g_in = sz[:, 2 * H:] + zr[:, 4 * H:] + zc[:, 4 * H:]
    cat3 = jnp.concatenate([_silu(a_in), _silu(b_in), _silu(g_in)], axis=1)
    abg = jnp.dot(cat3, wh[...], preferred_element_type=jnp.float32) + bh[...]
    ev = eav[:, 16:19]
    a = abg[:, 0:3]
    b = _sigm(abg[:, 3:6])
    g = jnp.tanh(abg[:, 6:9])
    apart = a * (ev * g)
    nb = ev.shape[0]
    out[...] = jnp.concatenate(
        [m_ij, apart, jnp.ones((nb, 1), jnp.float32), b,
         jnp.zeros((nb, D - H - 7), jnp.float32)], axis=1)


def _edge_call(ger, gec, eav, s8e, weights):
    full = lambda shape: pl.BlockSpec(shape, lambda i: (0, 0))
    wspecs = [full(w.shape) for w in weights]
    return pl.pallas_call(
        _edge_body,
        grid=(E // EB,),
        in_specs=[pl.BlockSpec((EB, H), lambda i: (i, 0)),
                  pl.BlockSpec((EB, H), lambda i: (i, 0)),
                  pl.BlockSpec((EB, 32), lambda i: (i, 0)),
                  pl.BlockSpec((EB, S8), lambda i: (i, 0))] + wspecs,
        name="edge",
        out_specs=pl.BlockSpec((EB, D), lambda i: (i, 0)),
        out_shape=jax.ShapeDtypeStruct((E, D), jnp.float32),
    )(ger, gec, eav, s8e, *weights)


def _node_body(p0, p1, h, pos, w0, b0, w1, b1, hout, posout):
    acc = p0[...] + p1[...]
    cnt = jnp.maximum(acc[:, H + 3:H + 4], 1.0)
    agg = acc[:, :H] / cnt
    dpos = (acc[:, H:H + 3] + acc[:, H + 4:H + 7] * pos[...]) / cnt
    hcat = jnp.concatenate([agg, h[...]], axis=1)
    u = _silu(jnp.dot(hcat, w0[...], preferred_element_type=jnp.float32)
              + b0[...])
    hout[...] = jnp.dot(u, w1[...], preferred_element_type=jnp.float32) \
        + b1[...] + h[...]
    posout[...] = pos[...] + dpos


def _node_call(parts, h, pos, w0, b0, w1, b1):
    nblk = N // NB
    full = lambda shape: pl.BlockSpec(shape, lambda i: (0, 0))
    return pl.pallas_call(
        _node_body,
        grid=(nblk,),
        in_specs=[pl.BlockSpec((NB, D), lambda i: (i, 0)),
                  pl.BlockSpec((NB, D), lambda i: (i + nblk, 0)),
                  pl.BlockSpec((NB, H), lambda i: (i, 0)),
                  pl.BlockSpec((NB, 3), lambda i: (i, 0)),
                  full(w0.shape), full(b0.shape),
                  full(w1.shape), full(b1.shape)],
        out_specs=[pl.BlockSpec((NB, H), lambda i: (i, 0)),
                   pl.BlockSpec((NB, 3), lambda i: (i, 0))],
        out_shape=[jax.ShapeDtypeStruct((N, H), jnp.float32),
                   jax.ShapeDtypeStruct((N, 3), jnp.float32)],
    )(parts, parts, h, pos, w0, b0, w1, b1)


def _dense_body(x, w, b, out):
    out[...] = jnp.dot(x[...], w[...],
                       preferred_element_type=jnp.float32) + b[...]


def _dense_call(x, w, b):
    n, di = x.shape
    do = w.shape[1]
    full = lambda shape: pl.BlockSpec(shape, lambda i: (0, 0))
    return pl.pallas_call(
        _dense_body,
        grid=(n // NB,),
        in_specs=[pl.BlockSpec((NB, di), lambda i: (i, 0)),
                  full(w.shape), full(b.shape)],
        out_specs=pl.BlockSpec((NB, do), lambda i: (i, 0)),
        out_shape=jax.ShapeDtypeStruct((n, do), jnp.float32),
    )(x, w, b)


def _final_body(h, w, b, hout, pooled):
    i = pl.program_id(0)
    hf = jnp.dot(h[...], w[...], preferred_element_type=jnp.float32) + b[...]
    hout[...] = hf
    s = jnp.sum(hf, axis=0, keepdims=True)

    @pl.when(i == 0)
    def _():
        pooled[...] = s

    @pl.when(i > 0)
    def _():
        pooled[...] += s

    @pl.when(i == (N // NB) - 1)
    def _():
        pooled[...] = pooled[...] * (1.0 / N)


def _final_call(h, w, b):
    full = lambda shape: pl.BlockSpec(shape, lambda i: (0, 0))
    return pl.pallas_call(
        _final_body,
        grid=(N // NB,),
        in_specs=[pl.BlockSpec((NB, H), lambda i: (i, 0)),
                  full(w.shape), full(b.shape)],
        out_specs=[pl.BlockSpec((NB, H), lambda i: (i, 0)),
                   pl.BlockSpec((1, H), lambda i: (0, 0))],
        out_shape=[jax.ShapeDtypeStruct((N, H), jnp.float32),
                   jax.ShapeDtypeStruct((1, H), jnp.float32)],
    )(h, w, b)


def _layer_weights(p):
    Qw, Qb = p["LnQ"]["w"], p["LnQ"]["b"]
    Kw, Kb = p["LnK"]["w"], p["LnK"]["b"]
    Vw, Vb = p["LnV"]["w"], p["LnV"]["b"]
    s0w, s0b = p["sigma0"]["w"], p["sigma0"]["b"]
    s0we, s0wk = s0w[:, :16], s0w[:, 16:]
    m0w, m0b = p["msg0"]["w"], p["msg0"]["b"]
    m0we, m0wr, m0wc = m0w[:, :16], m0w[:, 16:16 + H], m0w[:, 16 + H:]
    a0w, a0b = p["alpha0"]["w"], p["alpha0"]["b"]
    b0w, b0b = p["beta0"]["w"], p["beta0"]["b"]
    g0w, g0b = p["gate0"]["w"], p["gate0"]["b"]
    z = jnp.zeros((3 * H,), jnp.float32)
    W_r = jnp.concatenate([Qw, m0wr @ Vw, a0w[:, 6 + H:], b0w[:, 6 + H:],
                           g0w[:, 6 + H:]], axis=0)
    b_r = jnp.concatenate([Qb, m0wr @ Vb, z], axis=0)
    W_c = jnp.concatenate([s0wk @ Kw, m0wc @ Vw, a0w[:, 6:6 + H],
                           b0w[:, 6:6 + H], g0w[:, 6:6 + H]], axis=0)
    b_c = jnp.concatenate([s0wk @ Kb, m0wc @ Vb, a0b, b0b, g0b], axis=0)
    W_e2 = jnp.concatenate([s0we, m0we], axis=0)
    b_e2 = jnp.concatenate([s0b, m0b], axis=0)
    ws6 = jnp.concatenate([a0w[:, :6], b0w[:, :6], g0w[:, :6]], axis=0)
    W_s8 = jnp.concatenate([ws6, jnp.zeros((3 * H, 2), jnp.float32)], axis=1)
    zh = jnp.zeros((3, H), jnp.float32)
    Wh = jnp.concatenate([
        jnp.concatenate([p["alpha1"]["w"], zh, zh], axis=1),
        jnp.concatenate([zh, p["beta1"]["w"], zh], axis=1),
        jnp.concatenate([zh, zh, p["gate1"]["w"]], axis=1)], axis=0)
    bhd = jnp.concatenate([p["alpha1"]["b"], p["beta1"]["b"],
                           p["gate1"]["b"]], axis=0)
    r1 = lambda v: v.reshape(1, -1)
    bf = lambda v: v.astype(jnp.bfloat16)
    return [bf(W_r.T), r1(b_r), bf(W_c.T), r1(b_c),
            p["LnE"]["w"].T, r1(p["LnE"]["b"]),
            W_e2.T, r1(b_e2), W_s8.T,
            bf(p["sigma1"]["w"].T), r1(p["sigma1"]["b"]),
            bf(p["msg1"]["w"].T), r1(p["msg1"]["b"]),
            Wh.T, r1(bhd)]


def kernel(x, pos, edge_index, edge_attr, edge_vec, pos_frac, params):
    row = edge_index[0]
    col = edge_index[1]
    h = _dense_call(x, params["lin_in"]["w"].T,
                    params["lin_in"]["b"].reshape(1, -1))
    eav = jnp.concatenate(
        [edge_attr, edge_vec, jnp.zeros((E, 32 - 19), jnp.float32)], axis=1)
    pftab = jnp.concatenate(
        [pos_frac, jnp.zeros((N, SW - 3), jnp.float32)], axis=1)
    smr, smc = _sc_gather_pf(pftab, row, col)
    sel1 = np.zeros((96, S8), np.float32)
    sel2 = np.zeros((96, S8), np.float32)
    for j in range(3):
        sel1[16 * j:16 * (j + 1), j] = 1.0
        sel2[16 * (j + 3):16 * (j + 4), j + 3] = 1.0
    s8e = _prep_call(smr, smc, eav, jnp.asarray(sel1), jnp.asarray(sel2))
    zerosd = jnp.zeros((N, D), jnp.float32)
    for lp in params["layers"]:
        weights = _layer_weights(lp)
        ge_r, ge_c = _sc_gather_h(h, row, col)
        payload = _edge_call(ge_r, ge_c, eav, s8e, weights)
        parts = _sc_scatter(payload, col, zerosd)
        h, pos = _node_call(parts, h, pos,
                            lp["out0"]["w"].T, lp["out0"]["b"].reshape(1, -1),
                            lp["out1"]["w"].T, lp["out1"]["b"].reshape(1, -1))
    h_fin, pooled = _final_call(h, params["lin_out"]["w"].T,
                                params["lin_out"]["b"].reshape(1, -1))
    return h_fin, pooled


# speedup vs baseline: 1.4359x; 1.0819x over previous
import jax
import jax.numpy as jnp
import numpy as np
from jax import lax
from jax.experimental import pallas as pl
from jax.experimental.pallas import tpu as pltpu
from jax.experimental.pallas import tpu_sc as plsc

N = 10000
E = 320000
H = 128
SW = 16
S8 = 8
D = 144
NC = 2
NS = 16
NW = NC * NS
PER_W = E // NW
C = 80
NCH = PER_W // C
ROWS_PER_TILE = N // NS
RPT8 = 624

EB = 1600
NB = 1000

_INV_SQRT_H = float(1.0 / np.sqrt(float(H)))


def _sigm(v):
    return 0.5 * jnp.tanh(0.5 * v) + 0.5


def _silu(v):
    return v * _sigm(v)


_sc_mesh = plsc.VectorSubcoreMesh(core_axis_name="c", subcore_axis_name="s")


def _gather_h_body(htab, ridx, cidx, out_r, out_c,
                   idxr_v, idxc_v, bufr, bufc, semr, semc):
    wid = lax.axis_index("c") * NS + lax.axis_index("s")
    base = wid * PER_W

    def body(ck, carry):
        off = base + ck * C
        pltpu.sync_copy(ridx.at[pl.ds(off, C)], idxr_v)
        pltpu.sync_copy(cidx.at[pl.ds(off, C)], idxc_v)
        cp1 = pltpu.async_copy(htab.at[idxr_v], bufr, semr)
        cp2 = pltpu.async_copy(htab.at[idxc_v], bufc, semc)
        cp1.wait()
        cp2.wait()
        pltpu.sync_copy(bufr, out_r.at[pl.ds(off, C)])
        pltpu.sync_copy(bufc, out_c.at[pl.ds(off, C)])
        return carry

    lax.fori_loop(0, NCH, body, 0)


_sc_gather_h = pl.kernel(
    _gather_h_body,
    out_type=[jax.ShapeDtypeStruct((E, H), jnp.float32),
              jax.ShapeDtypeStruct((E, H), jnp.float32)],
    mesh=_sc_mesh,
    scratch_types=[
        pltpu.VMEM((C,), jnp.int32),
        pltpu.VMEM((C,), jnp.int32),
        pltpu.VMEM((C, H), jnp.float32),
        pltpu.VMEM((C, H), jnp.float32),
        pltpu.SemaphoreType.DMA,
        pltpu.SemaphoreType.DMA,
    ],
)


def _gather_pf_body(pftab, ridx, cidx, out_r, out_c,
                    idxr_v, idxc_v, bufr, bufc, semr, semc):
    wid = lax.axis_index("c") * NS + lax.axis_index("s")
    base = wid * PER_W

    def body(ck, carry):
        off = base + ck * C
        pltpu.sync_copy(ridx.at[pl.ds(off, C)], idxr_v)
        pltpu.sync_copy(cidx.at[pl.ds(off, C)], idxc_v)
        cp1 = pltpu.async_copy(pftab.at[idxr_v], bufr, semr)
        cp2 = pltpu.async_copy(pftab.at[idxc_v], bufc, semc)
        cp1.wait()
        cp2.wait()
        pltpu.sync_copy(bufr, out_r.at[pl.ds(off, C)])
        pltpu.sync_copy(bufc, out_c.at[pl.ds(off, C)])
        return carry

    lax.fori_loop(0, NCH, body, 0)


_sc_gather_pf = pl.kernel(
    _gather_pf_body,
    out_type=[jax.ShapeDtypeStruct((E, SW), jnp.float32),
              jax.ShapeDtypeStruct((E, SW), jnp.float32)],
    mesh=_sc_mesh,
    compiler_params=pltpu.CompilerParams(use_tc_tiling_on_sc=False),
    scratch_types=[
        pltpu.VMEM((C,), jnp.int32),
        pltpu.VMEM((C,), jnp.int32),
        pltpu.VMEM((C, SW), jnp.float32),
        pltpu.VMEM((C, SW), jnp.float32),
        pltpu.SemaphoreType.DMA,
        pltpu.SemaphoreType.DMA,
    ],
)


def _scatter_body(paym, pays, cidx, zerosm, zeross, outm, outs,
                  idx_v, paym_v, pays_v, accm, accs):
    cid = lax.axis_index("c")
    sid = lax.axis_index("s")
    wid = cid * NS + sid
    rows0 = sid * ROWS_PER_TILE
    pltpu.sync_copy(zerosm.at[pl.ds(rows0, ROWS_PER_TILE)],
                    accm.at[pl.ds(rows0, ROWS_PER_TILE)])
    pltpu.sync_copy(zeross.at[pl.ds(rows0, ROWS_PER_TILE)],
                    accs.at[pl.ds(rows0, ROWS_PER_TILE)])
    plsc.subcore_barrier()
    base = wid * PER_W

    def body(ck, carry):
        off = base + ck * C
        pltpu.sync_copy(cidx.at[pl.ds(off, C)], idx_v)
        pltpu.sync_copy(paym.at[pl.ds(off, C)], paym_v)
        pltpu.sync_copy(pays.at[pl.ds(off, C)], pays_v)
        pltpu.sync_copy(paym_v, accm.at[idx_v], add=True)
        pltpu.sync_copy(pays_v, accs.at[idx_v], add=True)
        return carry

    lax.fori_loop(0, NCH, body, 0)
    plsc.subcore_barrier()
    pltpu.sync_copy(accm.at[pl.ds(rows0, ROWS_PER_TILE)],
                    outm.at[pl.ds(cid * N + rows0, ROWS_PER_TILE)])
    pltpu.sync_copy(accs.at[pl.ds(rows0, ROWS_PER_TILE)],
                    outs.at[pl.ds(cid * N + rows0, ROWS_PER_TILE)])


_sc_scatter = pl.kernel(
    _scatter_body,
    out_type=[jax.ShapeDtypeStruct((2 * N, H), jnp.float32),
              jax.ShapeDtypeStruct((2 * N, SW), jnp.float32)],
    mesh=_sc_mesh,
    compiler_params=pltpu.CompilerParams(use_tc_tiling_on_sc=False),
    scratch_types=[
        pltpu.VMEM((C,), jnp.int32),
        pltpu.VMEM((C, H), jnp.float32),
        pltpu.VMEM((C, SW), jnp.float32),
        pltpu.VMEM_SHARED((N, H), jnp.float32),
        pltpu.VMEM_SHARED((N, SW), jnp.float32),
    ],
)


def _prep_body(smr, smc, eav, sel1, sel2, out):
    pr = smr[...]
    pc = smc[...]
    ev = eav[:, 16:32]
    prods = jnp.concatenate(
        [ev * pr, ev * pc, pr * pc, ev * ev, pr * pr, pc * pc], axis=1)
    s_lin = jnp.dot(prods, sel1[...], preferred_element_type=jnp.float32)
    s_sq = jnp.sqrt(jnp.dot(prods, sel2[...],
                            preferred_element_type=jnp.float32))
    out[...] = s_lin + s_sq


def _prep_call(smr, smc, eav, sel1, sel2):
    full = lambda shape: pl.BlockSpec(shape, lambda i: (0, 0))
    return pl.pallas_call(
        _prep_body,
        grid=(E // EB,),
        in_specs=[pl.BlockSpec((EB, SW), lambda i: (i, 0)),
                  pl.BlockSpec((EB, SW), lambda i: (i, 0)),
                  pl.BlockSpec((EB, 32), lambda i: (i, 0)),
                  full(sel1.shape), full(sel2.shape)],
        out_specs=pl.BlockSpec((EB, S8), lambda i: (i, 0)),
        out_shape=jax.ShapeDtypeStruct((E, S8), jnp.float32),
    )(smr, smc, eav, sel1, sel2)


def _edge_body(ger, gec, eav, s8e, wr, br, wc, bc, wle, ble, we2, be2, ws8,
               s1w, s1b, m1w, m1b, wh, bh, outm, outs):
    hr = ger[...].astype(jnp.bfloat16)
    hc = gec[...].astype(jnp.bfloat16)
    zr = jnp.dot(hr, wr[...], preferred_element_type=jnp.float32) + br[...]
    zc = jnp.dot(hc, wc[...], preferred_element_type=jnp.float32) + bc[...]
    ea = eav[:, :16]
    e = jnp.dot(ea, wle[...], preferred_element_type=jnp.float32) + ble[...]
    ez = jnp.dot(e, we2[...], preferred_element_type=jnp.float32) + be2[...]
    sig_in = ez[:, :H] + zc[:, :H]
    kk = _silu(jnp.dot(_silu(sig_in).astype(jnp.bfloat16), s1w[...],
                       preferred_element_type=jnp.float32) + s1b[...])
    q = zr[:, :H]
    w = _sigm(q * kk * _INV_SQRT_H)
    m_in = ez[:, H:] + zr[:, H:2 * H] + zc[:, H:2 * H]
    msg = _silu(jnp.dot(_silu(m_in).astype(jnp.bfloat16), m1w[...],
                        preferred_element_type=jnp.float32) + m1b[...])
    m_ij = w * msg

    sz = jnp.dot(s8e[...], ws8[...], preferred_element_type=jnp.float32)
    a_in = sz[:, :H] + zr[:, 2 * H:3 * H] + zc[:, 2 * H:3 * H]
    b_in = sz[:, H:2 * H] + zr[:, 3 * H:4 * H] + zc[:, 3 * H:4 * H]
    g_in = sz[:, 2 * H:] + zr[:, 4 * H:] + zc[:, 4 * H:]
    cat3 = jnp.concatenate([_silu(a_in), _silu(b_in), _silu(g_in)], axis=1)
    abg = jnp.dot(cat3, wh[...], preferred_element_type=jnp.float32) + bh[...]
    ev = eav[:, 16:19]
    a = abg[:, 0:3]
    b = _sigm(abg[:, 3:6])
    g = jnp.tanh(abg[:, 6:9])
    apart = a * (ev * g)
    nb = ev.shape[0]
    outm[...] = m_ij
    outs[...] = jnp.concatenate(
        [apart, jnp.ones((nb, 1), jnp.float32), b,
         jnp.zeros((nb, SW - 7), jnp.float32)], axis=1)


def _edge_call(ger, gec, eav, s8e, weights):
    full = lambda shape: pl.BlockSpec(shape, lambda i: (0, 0))
    wspecs = [full(w.shape) for w in weights]
    return pl.pallas_call(
        _edge_body,
        grid=(E // EB,),
        in_specs=[pl.BlockSpec((EB, H), lambda i: (i, 0)),
                  pl.BlockSpec((EB, H), lambda i: (i, 0)),
                  pl.BlockSpec((EB, 32), lambda i: (i, 0)),
                  pl.BlockSpec((EB, S8), lambda i: (i, 0))] + wspecs,
        name="edge",
        out_specs=[pl.BlockSpec((EB, H), lambda i: (i, 0)),
                   pl.BlockSpec((EB, SW), lambda i: (i, 0))],
        out_shape=[jax.ShapeDtypeStruct((E, H), jnp.float32),
                   jax.ShapeDtypeStruct((E, SW), jnp.float32)],
    )(ger, gec, eav, s8e, *weights)


def _node_body(p0, p1, s0, s1, h, pos, w0, b0, w1, b1, hout, posout):
    accm = p0[...] + p1[...]
    accs = s0[...] + s1[...]
    cnt = jnp.maximum(accs[:, 3:4], 1.0)
    agg = accm / cnt
    dpos = (accs[:, 0:3] + accs[:, 4:7] * pos[...]) / cnt
    hcat = jnp.concatenate([agg, h[...]], axis=1)
    u = _silu(jnp.dot(hcat, w0[...], preferred_element_type=jnp.float32)
              + b0[...])
    hout[...] = jnp.dot(u, w1[...], preferred_element_type=jnp.float32) \
        + b1[...] + h[...]
    posout[...] = pos[...] + dpos


def _node_call(partm, parts, h, pos, w0, b0, w1, b1):
    nblk = N // NB
    full = lambda shape: pl.BlockSpec(shape, lambda i: (0, 0))
    return pl.pallas_call(
        _node_body,
        grid=(nblk,),
        in_specs=[pl.BlockSpec((NB, H), lambda i: (i, 0)),
                  pl.BlockSpec((NB, H), lambda i: (i + nblk, 0)),
                  pl.BlockSpec((NB, SW), lambda i: (i, 0)),
                  pl.BlockSpec((NB, SW), lambda i: (i + nblk, 0)),
                  pl.BlockSpec((NB, H), lambda i: (i, 0)),
                  pl.BlockSpec((NB, 3), lambda i: (i, 0)),
                  full(w0.shape), full(b0.shape),
                  full(w1.shape), full(b1.shape)],
        out_specs=[pl.BlockSpec((NB, H), lambda i: (i, 0)),
                   pl.BlockSpec((NB, 3), lambda i: (i, 0))],
        out_shape=[jax.ShapeDtypeStruct((N, H), jnp.float32),
                   jax.ShapeDtypeStruct((N, 3), jnp.float32)],
    )(partm, partm, parts, parts, h, pos, w0, b0, w1, b1)


def _dense_body(x, w, b, out):
    out[...] = jnp.dot(x[...], w[...],
                       preferred_element_type=jnp.float32) + b[...]


def _dense_call(x, w, b):
    n, di = x.shape
    do = w.shape[1]
    full = lambda shape: pl.BlockSpec(shape, lambda i: (0, 0))
    return pl.pallas_call(
        _dense_body,
        grid=(n // NB,),
        in_specs=[pl.BlockSpec((NB, di), lambda i: (i, 0)),
                  full(w.shape), full(b.shape)],
        out_specs=pl.BlockSpec((NB, do), lambda i: (i, 0)),
        out_shape=jax.ShapeDtypeStruct((n, do), jnp.float32),
    )(x, w, b)


def _final_body(h, w, b, hout, pooled):
    i = pl.program_id(0)
    hf = jnp.dot(h[...], w[...], preferred_element_type=jnp.float32) + b[...]
    hout[...] = hf
    s = jnp.sum(hf, axis=0, keepdims=True)

    @pl.when(i == 0)
    def _():
        pooled[...] = s

    @pl.when(i > 0)
    def _():
        pooled[...] += s

    @pl.when(i == (N // NB) - 1)
    def _():
        pooled[...] = pooled[...] * (1.0 / N)


def _final_call(h, w, b):
    full = lambda shape: pl.BlockSpec(shape, lambda i: (0, 0))
    return pl.pallas_call(
        _final_body,
        grid=(N // NB,),
        in_specs=[pl.BlockSpec((NB, H), lambda i: (i, 0)),
                  full(w.shape), full(b.shape)],
        out_specs=[pl.BlockSpec((NB, H), lambda i: (i, 0)),
                   pl.BlockSpec((1, H), lambda i: (0, 0))],
        out_shape=[jax.ShapeDtypeStruct((N, H), jnp.float32),
                   jax.ShapeDtypeStruct((1, H), jnp.float32)],
    )(h, w, b)


def _layer_weights(p):
    Qw, Qb = p["LnQ"]["w"], p["LnQ"]["b"]
    Kw, Kb = p["LnK"]["w"], p["LnK"]["b"]
    Vw, Vb = p["LnV"]["w"], p["LnV"]["b"]
    s0w, s0b = p["sigma0"]["w"], p["sigma0"]["b"]
    s0we, s0wk = s0w[:, :16], s0w[:, 16:]
    m0w, m0b = p["msg0"]["w"], p["msg0"]["b"]
    m0we, m0wr, m0wc = m0w[:, :16], m0w[:, 16:16 + H], m0w[:, 16 + H:]
    a0w, a0b = p["alpha0"]["w"], p["alpha0"]["b"]
    b0w, b0b = p["beta0"]["w"], p["beta0"]["b"]
    g0w, g0b = p["gate0"]["w"], p["gate0"]["b"]
    z = jnp.zeros((3 * H,), jnp.float32)
    W_r = jnp.concatenate([Qw, m0wr @ Vw, a0w[:, 6 + H:], b0w[:, 6 + H:],
                           g0w[:, 6 + H:]], axis=0)
    b_r = jnp.concatenate([Qb, m0wr @ Vb, z], axis=0)
    W_c = jnp.concatenate([s0wk @ Kw, m0wc @ Vw, a0w[:, 6:6 + H],
                           b0w[:, 6:6 + H], g0w[:, 6:6 + H]], axis=0)
    b_c = jnp.concatenate([s0wk @ Kb, m0wc @ Vb, a0b, b0b, g0b], axis=0)
    W_e2 = jnp.concatenate([s0we, m0we], axis=0)
    b_e2 = jnp.concatenate([s0b, m0b], axis=0)
    ws6 = jnp.concatenate([a0w[:, :6], b0w[:, :6], g0w[:, :6]], axis=0)
    W_s8 = jnp.concatenate([ws6, jnp.zeros((3 * H, 2), jnp.float32)], axis=1)
    zh = jnp.zeros((3, H), jnp.float32)
    Wh = jnp.concatenate([
        jnp.concatenate([p["alpha1"]["w"], zh, zh], axis=1),
        jnp.concatenate([zh, p["beta1"]["w"], zh], axis=1),
        jnp.concatenate([zh, zh, p["gate1"]["w"]], axis=1)], axis=0)
    bhd = jnp.concatenate([p["alpha1"]["b"], p["beta1"]["b"],
                           p["gate1"]["b"]], axis=0)
    r1 = lambda v: v.reshape(1, -1)
    bf = lambda v: v.astype(jnp.bfloat16)
    return [bf(W_r.T), r1(b_r), bf(W_c.T), r1(b_c),
            p["LnE"]["w"].T, r1(p["LnE"]["b"]),
            W_e2.T, r1(b_e2), W_s8.T,
            bf(p["sigma1"]["w"].T), r1(p["sigma1"]["b"]),
            bf(p["msg1"]["w"].T), r1(p["msg1"]["b"]),
            Wh.T, r1(bhd)]


def kernel(x, pos, edge_index, edge_attr, edge_vec, pos_frac, params):
    row = edge_index[0]
    col = edge_index[1]
    h = _dense_call(x, params["lin_in"]["w"].T,
                    params["lin_in"]["b"].reshape(1, -1))
    eav = jnp.concatenate(
        [edge_attr, edge_vec, jnp.zeros((E, 32 - 19), jnp.float32)], axis=1)
    pftab = jnp.concatenate(
        [pos_frac, jnp.zeros((N, SW - 3), jnp.float32)], axis=1)
    smr, smc = _sc_gather_pf(pftab, row, col)
    sel1 = np.zeros((96, S8), np.float32)
    sel2 = np.zeros((96, S8), np.float32)
    for j in range(3):
        sel1[16 * j:16 * (j + 1), j] = 1.0
        sel2[16 * (j + 3):16 * (j + 4), j + 3] = 1.0
    s8e = _prep_call(smr, smc, eav, jnp.asarray(sel1), jnp.asarray(sel2))
    zerosm = jnp.zeros((N, H), jnp.float32)
    zeross = jnp.zeros((N, SW), jnp.float32)
    for lp in params["layers"]:
        weights = _layer_weights(lp)
        ge_r, ge_c = _sc_gather_h(h, row, col)
        paym, pays = _edge_call(ge_r, ge_c, eav, s8e, weights)
        partm, parts = _sc_scatter(paym, pays, col, zerosm, zeross)
        h, pos = _node_call(partm, parts, h, pos,
                            lp["out0"]["w"].T, lp["out0"]["b"].reshape(1, -1),
                            lp["out1"]["w"].T, lp["out1"]["b"].reshape(1, -1))
    h_fin, pooled = _final_call(h, params["lin_out"]["w"].T,
                                params["lin_out"]["b"].reshape(1, -1))
    return h_fin, pooled


# speedup vs baseline: 1.5533x; 1.0818x over previous
import jax
import jax.numpy as jnp
import numpy as np
from jax import lax
from jax.experimental import pallas as pl
from jax.experimental.pallas import tpu as pltpu
from jax.experimental.pallas import tpu_sc as plsc

N = 10000
E = 320000
H = 128
SW = 16
S8 = 8
D = 144
NC = 2
NS = 16
NW = NC * NS
PER_W = E // NW
C = 80
NCH = PER_W // C
ROWS_PER_TILE = N // NS
RPT8 = 624

EB = 1600
NB = 1000

_INV_SQRT_H = float(1.0 / np.sqrt(float(H)))


def _sigm(v):
    return 0.5 * jnp.tanh(0.5 * v) + 0.5


def _silu(v):
    return v * _sigm(v)


_sc_mesh = plsc.VectorSubcoreMesh(core_axis_name="c", subcore_axis_name="s")


def _gather_h_body(htab, ridx, cidx, out_r, out_c,
                   idxr_v, idxc_v, bufr, bufc, semr, semc):
    wid = lax.axis_index("c") * NS + lax.axis_index("s")
    base = wid * PER_W

    def body(ck, carry):
        off = base + ck * C
        pltpu.sync_copy(ridx.at[pl.ds(off, C)], idxr_v)
        pltpu.sync_copy(cidx.at[pl.ds(off, C)], idxc_v)
        cp1 = pltpu.async_copy(htab.at[idxr_v], bufr, semr)
        cp2 = pltpu.async_copy(htab.at[idxc_v], bufc, semc)
        cp1.wait()
        cp2.wait()
        pltpu.sync_copy(bufr, out_r.at[pl.ds(off, C)])
        pltpu.sync_copy(bufc, out_c.at[pl.ds(off, C)])
        return carry

    lax.fori_loop(0, NCH, body, 0)


_sc_gather_h = pl.kernel(
    _gather_h_body,
    out_type=[jax.ShapeDtypeStruct((E, H), jnp.float32),
              jax.ShapeDtypeStruct((E, H), jnp.float32)],
    mesh=_sc_mesh,
    scratch_types=[
        pltpu.VMEM((C,), jnp.int32),
        pltpu.VMEM((C,), jnp.int32),
        pltpu.VMEM((C, H), jnp.float32),
        pltpu.VMEM((C, H), jnp.float32),
        pltpu.SemaphoreType.DMA,
        pltpu.SemaphoreType.DMA,
    ],
)


def _gather_pf_body(pftab, ridx, cidx, out_r, out_c,
                    idxr_v, idxc_v, bufr, bufc, semr, semc):
    wid = lax.axis_index("c") * NS + lax.axis_index("s")
    base = wid * PER_W

    def body(ck, carry):
        off = base + ck * C
        pltpu.sync_copy(ridx.at[pl.ds(off, C)], idxr_v)
        pltpu.sync_copy(cidx.at[pl.ds(off, C)], idxc_v)
        cp1 = pltpu.async_copy(pftab.at[idxr_v], bufr, semr)
        cp2 = pltpu.async_copy(pftab.at[idxc_v], bufc, semc)
        cp1.wait()
        cp2.wait()
        pltpu.sync_copy(bufr, out_r.at[pl.ds(off, C)])
        pltpu.sync_copy(bufc, out_c.at[pl.ds(off, C)])
        return carry

    lax.fori_loop(0, NCH, body, 0)


_sc_gather_pf = pl.kernel(
    _gather_pf_body,
    out_type=[jax.ShapeDtypeStruct((E, SW), jnp.float32),
              jax.ShapeDtypeStruct((E, SW), jnp.float32)],
    mesh=_sc_mesh,
    compiler_params=pltpu.CompilerParams(use_tc_tiling_on_sc=False),
    scratch_types=[
        pltpu.VMEM((C,), jnp.int32),
        pltpu.VMEM((C,), jnp.int32),
        pltpu.VMEM((C, SW), jnp.float32),
        pltpu.VMEM((C, SW), jnp.float32),
        pltpu.SemaphoreType.DMA,
        pltpu.SemaphoreType.DMA,
    ],
)


def _scatter_body(paym, pays, cidx, zerosm, zeross, outm, outs,
                  idx_v, paym_v, pays_v, accm, accs):
    cid = lax.axis_index("c")
    sid = lax.axis_index("s")
    wid = cid * NS + sid
    rows0 = sid * ROWS_PER_TILE
    pltpu.sync_copy(zerosm.at[pl.ds(rows0, ROWS_PER_TILE)],
                    accm.at[pl.ds(rows0, ROWS_PER_TILE)])
    pltpu.sync_copy(zeross.at[pl.ds(rows0, ROWS_PER_TILE)],
                    accs.at[pl.ds(rows0, ROWS_PER_TILE)])
    plsc.subcore_barrier()
    base = wid * PER_W

    def body(ck, carry):
        off = base + ck * C
        pltpu.sync_copy(cidx.at[pl.ds(off, C)], idx_v)
        pltpu.sync_copy(paym.at[pl.ds(off, C)], paym_v)
        pltpu.sync_copy(pays.at[pl.ds(off, C)], pays_v)
        pltpu.sync_copy(paym_v, accm.at[idx_v], add=True)
        pltpu.sync_copy(pays_v, accs.at[idx_v], add=True)
        return carry

    lax.fori_loop(0, NCH, body, 0)
    plsc.subcore_barrier()
    pltpu.sync_copy(accm.at[pl.ds(rows0, ROWS_PER_TILE)],
                    outm.at[pl.ds(cid * N + rows0, ROWS_PER_TILE)])
    pltpu.sync_copy(accs.at[pl.ds(rows0, ROWS_PER_TILE)],
                    outs.at[pl.ds(cid * N + rows0, ROWS_PER_TILE)])


_sc_scatter = pl.kernel(
    _scatter_body,
    out_type=[jax.ShapeDtypeStruct((2 * N, H), jnp.float32),
              jax.ShapeDtypeStruct((2 * N, SW), jnp.float32)],
    mesh=_sc_mesh,
    compiler_params=pltpu.CompilerParams(use_tc_tiling_on_sc=False),
    scratch_types=[
        pltpu.VMEM((C,), jnp.int32),
        pltpu.VMEM((C, H), jnp.float32),
        pltpu.VMEM((C, SW), jnp.float32),
        pltpu.VMEM_SHARED((N, H), jnp.float32),
        pltpu.VMEM_SHARED((N, SW), jnp.float32),
    ],
)


def _prep_body(smr, smc, eav, sel1, sel2, out):
    pr = smr[...]
    pc = smc[...]
    ev = eav[:, 16:32]
    prods = jnp.concatenate(
        [ev * pr, ev * pc, pr * pc, ev * ev, pr * pr, pc * pc], axis=1)
    s_lin = jnp.dot(prods, sel1[...], preferred_element_type=jnp.float32)
    s_sq = jnp.sqrt(jnp.dot(prods, sel2[...],
                            preferred_element_type=jnp.float32))
    out[...] = s_lin + s_sq


def _prep_call(smr, smc, eav, sel1, sel2):
    PB = 8000
    full = lambda shape: pl.BlockSpec(shape, lambda i: (0, 0))
    return pl.pallas_call(
        _prep_body,
        grid=(E // PB,),
        in_specs=[pl.BlockSpec((PB, SW), lambda i: (i, 0)),
                  pl.BlockSpec((PB, SW), lambda i: (i, 0)),
                  pl.BlockSpec((PB, 32), lambda i: (i, 0)),
                  full(sel1.shape), full(sel2.shape)],
        out_specs=pl.BlockSpec((PB, S8), lambda i: (i, 0)),
        out_shape=jax.ShapeDtypeStruct((E, S8), jnp.float32),
    )(smr, smc, eav, sel1, sel2)


def _edge_body(ger, gec, eav, s8e, wcat, bcat, wle, ble, we2, be2, ws8,
               s1w, s1b, m1w, m1b, wh, bh, outm, outs):
    hcat = jnp.concatenate([ger[...], gec[...]], axis=1).astype(jnp.bfloat16)
    z = jnp.dot(hcat, wcat[...],
                preferred_element_type=jnp.float32) + bcat[...]
    ea = eav[:, :16]
    e = jnp.dot(ea, wle[...], preferred_element_type=jnp.float32) + ble[...]
    ez = jnp.dot(e, we2[...], preferred_element_type=jnp.float32) + be2[...]
    sig_in = ez[:, :H] + z[:, H:2 * H]
    kk = _silu(jnp.dot(_silu(sig_in).astype(jnp.bfloat16), s1w[...],
                       preferred_element_type=jnp.float32) + s1b[...])
    q = z[:, 0:H]
    w = _sigm(q * kk * _INV_SQRT_H)
    m_in = ez[:, H:] + z[:, 2 * H:3 * H]
    msg = _silu(jnp.dot(_silu(m_in).astype(jnp.bfloat16), m1w[...],
                        preferred_element_type=jnp.float32) + m1b[...])
    m_ij = w * msg

    sz = jnp.dot(s8e[...], ws8[...], preferred_element_type=jnp.float32)
    a_in = sz[:, :H] + z[:, 3 * H:4 * H]
    b_in = sz[:, H:2 * H] + z[:, 4 * H:5 * H]
    g_in = sz[:, 2 * H:] + z[:, 5 * H:6 * H]
    cat3 = jnp.concatenate([_silu(a_in), _silu(b_in), _silu(g_in)], axis=1)
    abg = jnp.dot(cat3, wh[...], preferred_element_type=jnp.float32) + bh[...]
    ev = eav[:, 16:19]
    a = abg[:, 0:3]
    b = _sigm(abg[:, 3:6])
    g = jnp.tanh(abg[:, 6:9])
    apart = a * (ev * g)
    nb = ev.shape[0]
    outm[...] = m_ij
    outs[...] = jnp.concatenate(
        [apart, jnp.ones((nb, 1), jnp.float32), b,
         jnp.zeros((nb, SW - 7), jnp.float32)], axis=1)


def _edge_call(ger, gec, eav, s8e, weights):
    full = lambda shape: pl.BlockSpec(shape, lambda i: (0, 0))
    wspecs = [full(w.shape) for w in weights]
    return pl.pallas_call(
        _edge_body,
        grid=(E // EB,),
        in_specs=[pl.BlockSpec((EB, H), lambda i: (i, 0)),
                  pl.BlockSpec((EB, H), lambda i: (i, 0)),
                  pl.BlockSpec((EB, 32), lambda i: (i, 0)),
                  pl.BlockSpec((EB, S8), lambda i: (i, 0))] + wspecs,
        name="edge",
        out_specs=[pl.BlockSpec((EB, H), lambda i: (i, 0)),
                   pl.BlockSpec((EB, SW), lambda i: (i, 0))],
        out_shape=[jax.ShapeDtypeStruct((E, H), jnp.float32),
                   jax.ShapeDtypeStruct((E, SW), jnp.float32)],
    )(ger, gec, eav, s8e, *weights)


def _node_body(p0, p1, s0, s1, h, pos, w0, b0, w1, b1, hout, posout):
    accm = p0[...] + p1[...]
    accs = s0[...] + s1[...]
    cnt = jnp.maximum(accs[:, 3:4], 1.0)
    agg = accm / cnt
    dpos = (accs[:, 0:3] + accs[:, 4:7] * pos[...]) / cnt
    hcat = jnp.concatenate([agg, h[...]], axis=1)
    u = _silu(jnp.dot(hcat, w0[...], preferred_element_type=jnp.float32)
              + b0[...])
    hout[...] = jnp.dot(u, w1[...], preferred_element_type=jnp.float32) \
        + b1[...] + h[...]
    posout[...] = pos[...] + dpos


def _node_call(partm, parts, h, pos, w0, b0, w1, b1):
    nblk = N // NB
    full = lambda shape: pl.BlockSpec(shape, lambda i: (0, 0))
    return pl.pallas_call(
        _node_body,
        grid=(nblk,),
        in_specs=[pl.BlockSpec((NB, H), lambda i: (i, 0)),
                  pl.BlockSpec((NB, H), lambda i: (i + nblk, 0)),
                  pl.BlockSpec((NB, SW), lambda i: (i, 0)),
                  pl.BlockSpec((NB, SW), lambda i: (i + nblk, 0)),
                  pl.BlockSpec((NB, H), lambda i: (i, 0)),
                  pl.BlockSpec((NB, 3), lambda i: (i, 0)),
                  full(w0.shape), full(b0.shape),
                  full(w1.shape), full(b1.shape)],
        out_specs=[pl.BlockSpec((NB, H), lambda i: (i, 0)),
                   pl.BlockSpec((NB, 3), lambda i: (i, 0))],
        out_shape=[jax.ShapeDtypeStruct((N, H), jnp.float32),
                   jax.ShapeDtypeStruct((N, 3), jnp.float32)],
    )(partm, partm, parts, parts, h, pos, w0, b0, w1, b1)


def _dense_body(x, w, b, out):
    out[...] = jnp.dot(x[...], w[...],
                       preferred_element_type=jnp.float32) + b[...]


def _dense_call(x, w, b):
    n, di = x.shape
    do = w.shape[1]
    full = lambda shape: pl.BlockSpec(shape, lambda i: (0, 0))
    return pl.pallas_call(
        _dense_body,
        grid=(n // NB,),
        in_specs=[pl.BlockSpec((NB, di), lambda i: (i, 0)),
                  full(w.shape), full(b.shape)],
        out_specs=pl.BlockSpec((NB, do), lambda i: (i, 0)),
        out_shape=jax.ShapeDtypeStruct((n, do), jnp.float32),
    )(x, w, b)


def _final_body(h, w, b, hout, pooled):
    i = pl.program_id(0)
    hf = jnp.dot(h[...], w[...], preferred_element_type=jnp.float32) + b[...]
    hout[...] = hf
    s = jnp.sum(hf, axis=0, keepdims=True)

    @pl.when(i == 0)
    def _():
        pooled[...] = s

    @pl.when(i > 0)
    def _():
        pooled[...] += s

    @pl.when(i == (N // NB) - 1)
    def _():
        pooled[...] = pooled[...] * (1.0 / N)


def _final_call(h, w, b):
    full = lambda shape: pl.BlockSpec(shape, lambda i: (0, 0))
    return pl.pallas_call(
        _final_body,
        grid=(N // NB,),
        in_specs=[pl.BlockSpec((NB, H), lambda i: (i, 0)),
                  full(w.shape), full(b.shape)],
        out_specs=[pl.BlockSpec((NB, H), lambda i: (i, 0)),
                   pl.BlockSpec((1, H), lambda i: (0, 0))],
        out_shape=[jax.ShapeDtypeStruct((N, H), jnp.float32),
                   jax.ShapeDtypeStruct((1, H), jnp.float32)],
    )(h, w, b)


def _layer_weights(p):
    Qw, Qb = p["LnQ"]["w"], p["LnQ"]["b"]
    Kw, Kb = p["LnK"]["w"], p["LnK"]["b"]
    Vw, Vb = p["LnV"]["w"], p["LnV"]["b"]
    s0w, s0b = p["sigma0"]["w"], p["sigma0"]["b"]
    s0we, s0wk = s0w[:, :16], s0w[:, 16:]
    m0w, m0b = p["msg0"]["w"], p["msg0"]["b"]
    m0we, m0wr, m0wc = m0w[:, :16], m0w[:, 16:16 + H], m0w[:, 16 + H:]
    a0w, a0b = p["alpha0"]["w"], p["alpha0"]["b"]
    b0w, b0b = p["beta0"]["w"], p["beta0"]["b"]
    g0w, g0b = p["gate0"]["w"], p["gate0"]["b"]
    zhh = jnp.zeros((H, H), jnp.float32)
    top = jnp.concatenate([Qw.T, zhh, (m0wr @ Vw).T, a0w[:, 6 + H:].T,
                           b0w[:, 6 + H:].T, g0w[:, 6 + H:].T], axis=1)
    bot = jnp.concatenate([zhh, (s0wk @ Kw).T, (m0wc @ Vw).T,
                           a0w[:, 6:6 + H].T, b0w[:, 6:6 + H].T,
                           g0w[:, 6:6 + H].T], axis=1)
    Wcat = jnp.concatenate([top, bot], axis=0)
    bcat = jnp.concatenate([Qb, s0wk @ Kb, (m0wr + m0wc) @ Vb,
                            a0b, b0b, g0b], axis=0)
    W_e2 = jnp.concatenate([s0we, m0we], axis=0)
    b_e2 = jnp.concatenate([s0b, m0b], axis=0)
    ws6 = jnp.concatenate([a0w[:, :6], b0w[:, :6], g0w[:, :6]], axis=0)
    W_s8 = jnp.concatenate([ws6, jnp.zeros((3 * H, 2), jnp.float32)], axis=1)
    zh = jnp.zeros((3, H), jnp.float32)
    Wh = jnp.concatenate([
        jnp.concatenate([p["alpha1"]["w"], zh, zh], axis=1),
        jnp.concatenate([zh, p["beta1"]["w"], zh], axis=1),
        jnp.concatenate([zh, zh, p["gate1"]["w"]], axis=1)], axis=0)
    bhd = jnp.concatenate([p["alpha1"]["b"], p["beta1"]["b"],
                           p["gate1"]["b"]], axis=0)
    r1 = lambda v: v.reshape(1, -1)
    bf = lambda v: v.astype(jnp.bfloat16)
    return [bf(Wcat), r1(bcat),
            p["LnE"]["w"].T, r1(p["LnE"]["b"]),
            W_e2.T, r1(b_e2), W_s8.T,
            bf(p["sigma1"]["w"].T), r1(p["sigma1"]["b"]),
            bf(p["msg1"]["w"].T), r1(p["msg1"]["b"]),
            Wh.T, r1(bhd)]


def kernel(x, pos, edge_index, edge_attr, edge_vec, pos_frac, params):
    row = edge_index[0]
    col = edge_index[1]
    h = _dense_call(x, params["lin_in"]["w"].T,
                    params["lin_in"]["b"].reshape(1, -1))
    eav = jnp.concatenate(
        [edge_attr, edge_vec, jnp.zeros((E, 32 - 19), jnp.float32)], axis=1)
    pftab = jnp.concatenate(
        [pos_frac, jnp.zeros((N, SW - 3), jnp.float32)], axis=1)
    smr, smc = _sc_gather_pf(pftab, row, col)
    sel1 = np.zeros((96, S8), np.float32)
    sel2 = np.zeros((96, S8), np.float32)
    for j in range(3):
        sel1[16 * j:16 * (j + 1), j] = 1.0
        sel2[16 * (j + 3):16 * (j + 4), j + 3] = 1.0
    s8e = _prep_call(smr, smc, eav, jnp.asarray(sel1), jnp.asarray(sel2))
    zerosm = jnp.zeros((N, H), jnp.float32)
    zeross = jnp.zeros((N, SW), jnp.float32)
    for lp in params["layers"]:
        weights = _layer_weights(lp)
        ge_r, ge_c = _sc_gather_h(h, row, col)
        paym, pays = _edge_call(ge_r, ge_c, eav, s8e, weights)
        partm, parts = _sc_scatter(paym, pays, col, zerosm, zeross)
        h, pos = _node_call(partm, parts, h, pos,
                            lp["out0"]["w"].T, lp["out0"]["b"].reshape(1, -1),
                            lp["out1"]["w"].T, lp["out1"]["b"].reshape(1, -1))
    h_fin, pooled = _final_call(h, params["lin_out"]["w"].T,
                                params["lin_out"]["b"].reshape(1, -1))
    return h_fin, pooled
